# Initial kernel scaffold; baseline (speedup 1.0000x reference)
import jax
import jax.numpy as jnp
from jax.experimental import pallas as pl


def kernel(tokens, emb, gru_wih, gru_whh, gru_bih, gru_bhh, ctrl_w1, ctrl_b1, ctrl_w2, ctrl_b2, gate_w, gate_b, assign_w, assign_b, nov_w, nov_b, relax_w, relax_b, cc_w, cc_b, cs_w, cs_b, md_w, md_b, att_w, att_b, out_w1, out_b1, out_w2, out_b2):
    raise NotImplementedError("write your pallas kernel here")


# speedup vs baseline: 10.2370x; 10.2370x over previous
import functools
import math

import jax
import jax.numpy as jnp
from jax import lax
from jax.experimental import pallas as pl
from jax.experimental.pallas import tpu as pltpu

_V, _C, _H, _E = 32000, 32, 256, 256
_B, _S = 8, 128
_EPS = 1e-4
_BC = _B * _C
_NEG = -1e30
_INVSQRT2 = 1.0 / math.sqrt(2.0)


def _dot(a, b):
    return jnp.dot(a, b, preferred_element_type=jnp.float32)


def _dg(a, b, dims):
    return lax.dot_general(a, b, (dims, ((), ())),
                           preferred_element_type=jnp.float32)


def _split(a):
    hi = a.astype(jnp.bfloat16).astype(jnp.float32)
    return hi, a - hi


def _xl(a, b):
    hi, lo = _split(a)
    return _dot(hi, b) + _dot(lo, b)


def _xr(a, b):
    hi, lo = _split(b)
    return _dot(a, hi) + _dot(a, lo)


def _consts():
    f32 = jnp.float32
    i32 = jnp.int32
    bmask8 = (lax.broadcasted_iota(i32, (_B, _BC), 1) % _B
              == lax.broadcasted_iota(i32, (_B, _BC), 0)).astype(f32)
    Q = (lax.broadcasted_iota(i32, (_C, _BC), 1) // _B
         == lax.broadcasted_iota(i32, (_C, _BC), 0)).astype(f32)
    P = (lax.broadcasted_iota(i32, (_BC, _C), 0) // _B
         == lax.broadcasted_iota(i32, (_BC, _C), 1)).astype(f32)
    eexp = (lax.broadcasted_iota(i32, (_BC, _B), 0) % _B
            == lax.broadcasted_iota(i32, (_BC, _B), 1)).astype(f32)
    eye = (lax.broadcasted_iota(i32, (_BC, _BC), 0)
           == lax.broadcasted_iota(i32, (_BC, _BC), 1)).astype(f32)
    same = (lax.broadcasted_iota(i32, (_BC, _BC), 0) % _B
            == lax.broadcasted_iota(i32, (_BC, _BC), 1) % _B)
    return bmask8, Q, P, eexp, eye, same


def _to_row(m, Q, bmask8):
    return jnp.sum(_xl(m, Q) * bmask8, axis=0, keepdims=True)


def _row_to_col(r, eye):
    return _dg(eye, _split(r)[0], ((1,), (1,))) + \
        _dg(eye, r - _split(r)[0], ((1,), (1,)))


def _col_to_row(c, eye):
    hi, lo = _split(c)
    return _dg(hi, eye, ((0,), (0,))) + _dg(lo, eye, ((0,), (0,)))


def _row_to_m(r, P, bmask8):
    return _xl(jnp.broadcast_to(r, (_B, _BC)) * bmask8, P)


def _summarize(c2d, sp, ma, bmask8, Q, eexp, eye):
    prec = 1.0 / (sp + _EPS)
    scores = ma + jnp.log(prec + _EPS)
    smax = jnp.max(scores, axis=-1, keepdims=True)
    e = jnp.exp(scores - smax)
    alpha = e / jnp.sum(e, axis=-1, keepdims=True)
    A = _xl(alpha, Q) * bmask8
    core = _dot(A, c2d)
    unc = jnp.sum(alpha * sp, axis=-1, keepdims=True)
    core_exp = _xr(eexp, core)
    sqd = jnp.mean((c2d - core_exp) ** 2, axis=-1, keepdims=True)
    sqd_row = _col_to_row(sqd, eye)
    div = jnp.sum(A * sqd_row, axis=-1, keepdims=True)
    mmax = jnp.max(ma, axis=-1, keepdims=True)
    en = jnp.log(jnp.sum(jnp.exp(ma - mmax), axis=-1, keepdims=True)) + mmax
    ent = -jnp.sum(alpha * jnp.log(jnp.maximum(alpha, 1e-8)),
                   axis=-1, keepdims=True)
    return core, unc, div, en, ent, alpha


def _scan_kernel(x_ref, mask_ref, wihT_ref, bih_ref, whhT_ref, bhh_ref,
                 w1lc_ref, w1s_ref, b1_ref, w2_ref, b2_ref,
                 hw_ref, hb_ref, ccw_ref, ccb_ref, attw_ref, attb_ref,
                 o1m_ref, o1s_ref, ob1_ref,
                 h1_ref,
                 c2d_ref, sp_ref, ma_ref, h_ref, gi_ref, cnd_ref):
    s = pl.program_id(0)

    @pl.when(s == 0)
    def _init():
        c2d_ref[...] = jnp.zeros_like(c2d_ref)
        sp_ref[...] = jnp.ones_like(sp_ref)
        ma_ref[...] = jnp.zeros_like(ma_ref)
        h_ref[...] = jnp.zeros_like(h_ref)
        gi_ref[...] = _dot(x_ref[...], wihT_ref[...]) + bih_ref[...]

    bmask8, Q, P, eexp, eye, same = _consts()
    valid = mask_ref[0]

    h = h_ref[...]
    gi = gi_ref[pl.ds(pl.multiple_of(s * _B, _B), _B), :]
    gh = _dot(h, whhT_ref[...]) + bhh_ref[...]
    r = jax.nn.sigmoid(gi[:, :_H] + gh[:, :_H])
    z = jax.nn.sigmoid(gi[:, _H:2 * _H] + gh[:, _H:2 * _H])
    n = jnp.tanh(gi[:, 2 * _H:] + r * gh[:, 2 * _H:])
    h = (1.0 - z) * n + z * h
    h_ref[...] = h
    local = h

    c2d = c2d_ref[...]
    sp = sp_ref[...]
    ma = ma_ref[...]

    core, unc, div, en, ent, _ = _summarize(c2d, sp, ma, bmask8, Q, eexp, eye)
    pre = _dot(jnp.concatenate([local, core], axis=1), w1lc_ref[...]) + b1_ref[...]
    pre = (pre + unc * w1s_ref[0:1, :] + div * w1s_ref[1:2, :]
           + en * w1s_ref[2:3, :] + ent * w1s_ref[3:4, :])
    ctrl = jnp.tanh(_dot(jnp.tanh(pre), w2_ref[...]) + b2_ref[...])

    hs = _dot(ctrl, hw_ref[...]) + hb_ref[...]
    gate = jax.nn.sigmoid(hs[:, 0:_C]) * valid
    ae = hs[:, _C:2 * _C]
    ae = jnp.exp(ae - jnp.max(ae, axis=-1, keepdims=True))
    assign = ae / jnp.sum(ae, axis=-1, keepdims=True)
    cs_raw = hs[:, 2 * _C:3 * _C]
    cand_sp = (jnp.maximum(cs_raw, 0.0)
               + jnp.log(1.0 + jnp.exp(-jnp.abs(cs_raw))) + _EPS)
    mdel = jnp.tanh(hs[:, 3 * _C:4 * _C])
    nov = jax.nn.sigmoid(hs[:, 4 * _C:4 * _C + 1]) * valid
    relax = jax.nn.sigmoid(hs[:, 4 * _C + 1:4 * _C + 2]) * valid

    cand = _dot(ctrl, ccw_ref[...]) + ccb_ref[...]
    for c in range(_C):
        cnd_ref[c * _B:(c + 1) * _B, :] = cand[:, c * _H:(c + 1) * _H]
    cand2d = cnd_ref[...]

    strength = gate * assign
    st_col = _row_to_col(_to_row(strength, Q, bmask8), eye)
    c2d = c2d + st_col * (cand2d - c2d)
    sp = sp + strength * (cand_sp - sp)
    ma = ma + strength * mdel

    attr = _dot(ctrl, attw_ref[...]) + attb_ref[...]
    anr = _xr(eexp, jnp.concatenate([attr, nov, relax], axis=1))
    attr_exp = anr[:, :_H]
    nov_col = anr[:, _H:_H + 1]
    relax_col = anr[:, _H + 1:_H + 2]
    c2d = c2d + 0.1 * nov_col * (attr_exp - c2d)

    sp_row = _to_row(sp, Q, bmask8)
    ma_row = _to_row(ma, Q, bmask8)
    sp_col = _row_to_col(sp_row, eye)
    sq_col = jnp.sum(c2d * c2d, axis=-1, keepdims=True)
    sq_row = _col_to_row(sq_col, eye)
    G = _dg(c2d, c2d, ((1,), (1,)))
    d2 = jnp.maximum(sq_col + sq_row - 2.0 * G, 0.0)
    scale = sp_col + sp_row + _EPS
    compat = jnp.where(same, -d2 / scale + ma_row, _NEG)
    cmax = jnp.max(compat, axis=-1, keepdims=True)
    cexp = jnp.exp(compat - cmax)
    mixing = cexp / jnp.sum(cexp, axis=-1, keepdims=True)
    mc = _dot(mixing, c2d)
    msp = jnp.sum(mixing * sp_row, axis=-1, keepdims=True)
    mma = jnp.sum(mixing * ma_row, axis=-1, keepdims=True)

    c2d = (1.0 - relax_col) * c2d + relax_col * mc
    sp = (1.0 - relax) * sp + relax * _row_to_m(_col_to_row(msp, eye), P, bmask8)
    ma = (1.0 - relax) * ma + relax * _row_to_m(_col_to_row(mma, eye), P, bmask8)

    c2d_ref[...] = c2d
    sp_ref[...] = sp
    ma_ref[...] = ma

    core2, unc2, div2, en2, ent2, alpha2 = _summarize(
        c2d, sp, ma, bmask8, Q, eexp, eye)
    cidx = lax.broadcasted_iota(jnp.int32, (_B, _C), 1)
    amax = jnp.max(alpha2, axis=-1, keepdims=True)
    idx = jnp.min(jnp.where(alpha2 == amax, cidx, _C), axis=-1, keepdims=True)
    oh = _dot((cidx == idx).astype(jnp.float32), Q) * bmask8
    strongest = _xr(oh, c2d)

    feat = jnp.concatenate([local, core2, strongest], axis=1)
    h1 = _dot(feat, o1m_ref[...]) + ob1_ref[...]
    h1 = (h1 + unc2 * o1s_ref[0:1, :] + div2 * o1s_ref[1:2, :]
          + en2 * o1s_ref[2:3, :] + ent2 * o1s_ref[3:4, :])
    h1 = 0.5 * h1 * (1.0 + lax.erf(h1 * _INVSQRT2))
    h1_ref[...] = h1


def _logits_kernel(a_ref, w_ref, b_ref, o_ref):
    o_ref[...] = _dot(a_ref[...], w_ref[...]) + b_ref[...]


_NT = 1280


@functools.partial(jax.jit, static_argnames=("interpret",))
def _run(x_flat, maskS, wihT, bih2, whhT, bhh2, w1lc, w1s, b1_2, ctrl_w2,
         b2_2, hw, hb, cc_w, cc_b2, att_w, attb2, o1m, o1s, ob1, out_w2,
         ob2, interpret=False):
    const = lambda s: (0, 0)
    h1_sb = pl.pallas_call(
        _scan_kernel,
        out_shape=jax.ShapeDtypeStruct((_S * _B, _H), jnp.float32),
        grid=(_S,),
        in_specs=[
            pl.BlockSpec((_S * _B, _E), const),
            pl.BlockSpec((1, _B, 1), lambda s: (s, 0, 0)),
            pl.BlockSpec((_E, 3 * _H), const),
            pl.BlockSpec((1, 3 * _H), const),
            pl.BlockSpec((_H, 3 * _H), const),
            pl.BlockSpec((1, 3 * _H), const),
            pl.BlockSpec((2 * _H, _H), const),
            pl.BlockSpec((4, _H), const),
            pl.BlockSpec((1, _H), const),
            pl.BlockSpec((_H, _H), const),
            pl.BlockSpec((1, _H), const),
            pl.BlockSpec((_H, 4 * _C + 2), const),
            pl.BlockSpec((1, 4 * _C + 2), const),
            pl.BlockSpec((_H, _C * _H), const),
            pl.BlockSpec((1, _C * _H), const),
            pl.BlockSpec((_H, _H), const),
            pl.BlockSpec((1, _H), const),
            pl.BlockSpec((3 * _H, _H), const),
            pl.BlockSpec((4, _H), const),
            pl.BlockSpec((1, _H), const),
        ],
        out_specs=pl.BlockSpec((_B, _H), lambda s: (s, 0)),
        scratch_shapes=[
            pltpu.VMEM((_BC, _H), jnp.float32),
            pltpu.VMEM((_B, _C), jnp.float32),
            pltpu.VMEM((_B, _C), jnp.float32),
            pltpu.VMEM((_B, _H), jnp.float32),
            pltpu.VMEM((_S * _B, 3 * _H), jnp.float32),
            pltpu.VMEM((_BC, _H), jnp.float32),
        ],
        compiler_params=pltpu.CompilerParams(
            dimension_semantics=("arbitrary",),
            vmem_limit_bytes=50 * 1024 * 1024,
        ),
        name="cfrm_scan",
        interpret=interpret,
    )(x_flat, maskS, wihT, bih2, whhT, bhh2, w1lc, w1s, b1_2, ctrl_w2,
      b2_2, hw, hb, cc_w, cc_b2, att_w, attb2, o1m, o1s, ob1)

    h1_bs = jnp.swapaxes(h1_sb.reshape(_S, _B, _H), 0, 1).reshape(_B * _S, _H)
    logits = pl.pallas_call(
        _logits_kernel,
        out_shape=jax.ShapeDtypeStruct((_B * _S, _V), jnp.float32),
        grid=(_V // _NT,),
        in_specs=[
            pl.BlockSpec((_B * _S, _H), lambda j: (0, 0)),
            pl.BlockSpec((_H, _NT), lambda j: (0, j)),
            pl.BlockSpec((1, _NT), lambda j: (0, j)),
        ],
        out_specs=pl.BlockSpec((_B * _S, _NT), lambda j: (0, j)),
        compiler_params=pltpu.CompilerParams(
            dimension_semantics=("parallel",),
        ),
        name="cfrm_logits",
        interpret=interpret,
    )(h1_bs, out_w2, ob2)
    return logits.reshape(_B, _S, _V)


def kernel(tokens, emb, gru_wih, gru_whh, gru_bih, gru_bhh, ctrl_w1, ctrl_b1,
           ctrl_w2, ctrl_b2, gate_w, gate_b, assign_w, assign_b, nov_w, nov_b,
           relax_w, relax_b, cc_w, cc_b, cs_w, cs_b, md_w, md_b, att_w, att_b,
           out_w1, out_b1, out_w2, out_b2, *, interpret=False):
    x = emb[tokens]
    x_flat = jnp.swapaxes(x, 0, 1).reshape(_S * _B, _E)
    maskS = (tokens != 0).astype(jnp.float32).T[:, :, None]
    hw = jnp.concatenate([gate_w, assign_w, cs_w, md_w, nov_w, relax_w], axis=1)
    hb = jnp.concatenate([gate_b, assign_b, cs_b, md_b, nov_b, relax_b])[None, :]
    return _run(x_flat, maskS, gru_wih.T, gru_bih[None, :], gru_whh.T,
                gru_bhh[None, :], ctrl_w1[:2 * _H], ctrl_w1[2 * _H:],
                ctrl_b1[None, :], ctrl_w2, ctrl_b2[None, :], hw, hb,
                cc_w, cc_b[None, :], att_w, att_b[None, :],
                out_w1[:3 * _H], out_w1[3 * _H:], out_b1[None, :],
                out_w2, out_b2[None, :], interpret=interpret)


# speedup vs baseline: 11.2984x; 1.1037x over previous
import functools
import math

import jax
import jax.numpy as jnp
from jax import lax
from jax.experimental import pallas as pl
from jax.experimental.pallas import tpu as pltpu

_V, _C, _H, _E = 32000, 32, 256, 256
_B, _S = 8, 128
_EPS = 1e-4
_BC = _B * _C
_NEG = -1e30
_INVSQRT2 = 1.0 / math.sqrt(2.0)


def _dot(a, b):
    return jnp.dot(a, b, preferred_element_type=jnp.float32)


def _dg(a, b, dims):
    return lax.dot_general(a, b, (dims, ((), ())),
                           preferred_element_type=jnp.float32)


def _split(a):
    hi = a.astype(jnp.bfloat16).astype(jnp.float32)
    return hi, a - hi


def _xl(a, b):
    hi, lo = _split(a)
    return _dot(hi, b) + _dot(lo, b)


def _xr(a, b):
    hi, lo = _split(b)
    return _dot(a, hi) + _dot(a, lo)


def _consts():
    f32 = jnp.float32
    i32 = jnp.int32
    bmask8 = (lax.broadcasted_iota(i32, (_B, _BC), 1) % _B
              == lax.broadcasted_iota(i32, (_B, _BC), 0)).astype(f32)
    Q = (lax.broadcasted_iota(i32, (_C, _BC), 1) // _B
         == lax.broadcasted_iota(i32, (_C, _BC), 0)).astype(f32)
    P = (lax.broadcasted_iota(i32, (_BC, _C), 0) // _B
         == lax.broadcasted_iota(i32, (_BC, _C), 1)).astype(f32)
    eexp = (lax.broadcasted_iota(i32, (_BC, _B), 0) % _B
            == lax.broadcasted_iota(i32, (_BC, _B), 1)).astype(f32)
    eye = (lax.broadcasted_iota(i32, (_BC, _BC), 0)
           == lax.broadcasted_iota(i32, (_BC, _BC), 1)).astype(f32)
    same = (lax.broadcasted_iota(i32, (_BC, _BC), 0) % _B
            == lax.broadcasted_iota(i32, (_BC, _BC), 1) % _B)
    return bmask8, Q, P, eexp, eye, same


def _to_row(m, Q, bmask8):
    return jnp.sum(_xl(m, Q) * bmask8, axis=0, keepdims=True)


def _row_to_col(r, eye):
    return _dg(eye, _split(r)[0], ((1,), (1,))) + \
        _dg(eye, r - _split(r)[0], ((1,), (1,)))


def _col_to_row(c, eye):
    hi, lo = _split(c)
    return _dg(hi, eye, ((0,), (0,))) + _dg(lo, eye, ((0,), (0,)))


def _row_to_m(r, P, bmask8):
    return _xl(jnp.broadcast_to(r, (_B, _BC)) * bmask8, P)


def _summarize(c2d, sp, ma, bmask8, Q, eexp, eye):
    prec = 1.0 / (sp + _EPS)
    scores = ma + jnp.log(prec + _EPS)
    smax = jnp.max(scores, axis=-1, keepdims=True)
    e = jnp.exp(scores - smax)
    alpha = e / jnp.sum(e, axis=-1, keepdims=True)
    A = _xl(alpha, Q) * bmask8
    core = _dot(A, c2d)
    unc = jnp.sum(alpha * sp, axis=-1, keepdims=True)
    core_exp = _xr(eexp, core)
    sqd = jnp.mean((c2d - core_exp) ** 2, axis=-1, keepdims=True)
    sqd_row = _col_to_row(sqd, eye)
    div = jnp.sum(A * sqd_row, axis=-1, keepdims=True)
    mmax = jnp.max(ma, axis=-1, keepdims=True)
    en = jnp.log(jnp.sum(jnp.exp(ma - mmax), axis=-1, keepdims=True)) + mmax
    ent = -jnp.sum(alpha * jnp.log(jnp.maximum(alpha, 1e-8)),
                   axis=-1, keepdims=True)
    return core, unc, div, en, ent, alpha


def _scan_kernel(gi_ref, mask_ref, whhT_ref, bhh_ref,
                 w1lc_ref, w1s_ref, b1_ref, w2_ref, b2_ref,
                 hw_ref, hb_ref, ccw_ref, ccb_ref, attw_ref, attb_ref,
                 o1m_ref, o1s_ref, ob1_ref,
                 h1_ref,
                 c2d_ref, sp_ref, ma_ref, h_ref, cnd_ref):
    s = pl.program_id(0)

    @pl.when(s == 0)
    def _init():
        c2d_ref[...] = jnp.zeros_like(c2d_ref)
        sp_ref[...] = jnp.ones_like(sp_ref)
        ma_ref[...] = jnp.zeros_like(ma_ref)
        h_ref[...] = jnp.zeros_like(h_ref)

    bmask8, Q, P, eexp, eye, same = _consts()
    valid = mask_ref[0]

    h = h_ref[...]
    gi = gi_ref[pl.ds(pl.multiple_of(s * _B, _B), _B), :]
    gh = _dot(h.astype(jnp.bfloat16), whhT_ref[...]) + bhh_ref[...]
    r = jax.nn.sigmoid(gi[:, :_H] + gh[:, :_H])
    z = jax.nn.sigmoid(gi[:, _H:2 * _H] + gh[:, _H:2 * _H])
    n = jnp.tanh(gi[:, 2 * _H:] + r * gh[:, 2 * _H:])
    h = (1.0 - z) * n + z * h
    h_ref[...] = h
    local = h

    c2d = c2d_ref[...]
    sp = sp_ref[...]
    ma = ma_ref[...]

    core, unc, div, en, ent, _ = _summarize(c2d, sp, ma, bmask8, Q, eexp, eye)
    pre = _dot(jnp.concatenate([local, core], axis=1).astype(jnp.bfloat16),
               w1lc_ref[...]) + b1_ref[...]
    pre = (pre + unc * w1s_ref[0:1, :] + div * w1s_ref[1:2, :]
           + en * w1s_ref[2:3, :] + ent * w1s_ref[3:4, :])
    ctrl = jnp.tanh(_dot(jnp.tanh(pre).astype(jnp.bfloat16), w2_ref[...])
                    + b2_ref[...])
    ctrl_bf = ctrl.astype(jnp.bfloat16)

    hs = _dot(ctrl_bf, hw_ref[...]) + hb_ref[...]
    gate = jax.nn.sigmoid(hs[:, 0:_C]) * valid
    ae = hs[:, _C:2 * _C]
    ae = jnp.exp(ae - jnp.max(ae, axis=-1, keepdims=True))
    assign = ae / jnp.sum(ae, axis=-1, keepdims=True)
    cs_raw = hs[:, 2 * _C:3 * _C]
    cand_sp = (jnp.maximum(cs_raw, 0.0)
               + jnp.log(1.0 + jnp.exp(-jnp.abs(cs_raw))) + _EPS)
    mdel = jnp.tanh(hs[:, 3 * _C:4 * _C])
    nov = jax.nn.sigmoid(hs[:, 4 * _C:4 * _C + 1]) * valid
    relax = jax.nn.sigmoid(hs[:, 4 * _C + 1:4 * _C + 2]) * valid

    cand = _dot(ctrl_bf, ccw_ref[...]) + ccb_ref[...]
    for c in range(_C):
        cnd_ref[c * _B:(c + 1) * _B, :] = cand[:, c * _H:(c + 1) * _H]
    cand2d = cnd_ref[...]

    strength = gate * assign
    st_col = _row_to_col(_to_row(strength, Q, bmask8), eye)
    c2d = c2d + st_col * (cand2d - c2d)
    sp = sp + strength * (cand_sp - sp)
    ma = ma + strength * mdel

    attr = _dot(ctrl_bf, attw_ref[...]) + attb_ref[...]
    anr = _xr(eexp, jnp.concatenate([attr, nov, relax], axis=1))
    attr_exp = anr[:, :_H]
    nov_col = anr[:, _H:_H + 1]
    relax_col = anr[:, _H + 1:_H + 2]
    c2d = c2d + 0.1 * nov_col * (attr_exp - c2d)

    sp_row = _to_row(sp, Q, bmask8)
    ma_row = _to_row(ma, Q, bmask8)
    sp_col = _row_to_col(sp_row, eye)
    sq_col = jnp.sum(c2d * c2d, axis=-1, keepdims=True)
    sq_row = _col_to_row(sq_col, eye)
    G = _dg(c2d, c2d, ((1,), (1,)))
    d2 = jnp.maximum(sq_col + sq_row - 2.0 * G, 0.0)
    scale = sp_col + sp_row + _EPS
    compat = jnp.where(same, -d2 / scale + ma_row, _NEG)
    cmax = jnp.max(compat, axis=-1, keepdims=True)
    cexp = jnp.exp(compat - cmax)
    mixing = cexp / jnp.sum(cexp, axis=-1, keepdims=True)
    mc = _dot(mixing, c2d)
    msp = jnp.sum(mixing * sp_row, axis=-1, keepdims=True)
    mma = jnp.sum(mixing * ma_row, axis=-1, keepdims=True)

    c2d = (1.0 - relax_col) * c2d + relax_col * mc
    sp = (1.0 - relax) * sp + relax * _row_to_m(_col_to_row(msp, eye), P, bmask8)
    ma = (1.0 - relax) * ma + relax * _row_to_m(_col_to_row(mma, eye), P, bmask8)

    c2d_ref[...] = c2d
    sp_ref[...] = sp
    ma_ref[...] = ma

    core2, unc2, div2, en2, ent2, alpha2 = _summarize(
        c2d, sp, ma, bmask8, Q, eexp, eye)
    cidx = lax.broadcasted_iota(jnp.int32, (_B, _C), 1)
    amax = jnp.max(alpha2, axis=-1, keepdims=True)
    idx = jnp.min(jnp.where(alpha2 == amax, cidx, _C), axis=-1, keepdims=True)
    oh = _dot((cidx == idx).astype(jnp.float32), Q) * bmask8
    strongest = _xr(oh, c2d)

    feat = jnp.concatenate([local, core2, strongest], axis=1)
    h1 = _dot(feat.astype(jnp.bfloat16), o1m_ref[...]) + ob1_ref[...]
    h1 = (h1 + unc2 * o1s_ref[0:1, :] + div2 * o1s_ref[1:2, :]
          + en2 * o1s_ref[2:3, :] + ent2 * o1s_ref[3:4, :])
    h1 = 0.5 * h1 * (1.0 + lax.erf(h1 * _INVSQRT2))
    h1_ref[...] = h1


def _logits_kernel(a_ref, w_ref, b_ref, o_ref):
    o_ref[...] = _dot(a_ref[...], w_ref[...]) + b_ref[...]


_NT = 1280


@functools.partial(jax.jit, static_argnames=("interpret",))
def _run(x_flat, maskS, wihT, bih2, whhT, bhh2, w1lc, w1s, b1_2, ctrl_w2,
         b2_2, hw, hb, cc_w, cc_b2, att_w, attb2, o1m, o1s, ob1, out_w2,
         ob2, interpret=False):
    const = lambda s: (0, 0)
    bf = jnp.bfloat16
    gi_all = pl.pallas_call(
        _logits_kernel,
        out_shape=jax.ShapeDtypeStruct((_S * _B, 3 * _H), jnp.float32),
        name="cfrm_gi",
        interpret=interpret,
    )(x_flat.astype(bf), wihT.astype(bf), bih2)

    h1_sb = pl.pallas_call(
        _scan_kernel,
        out_shape=jax.ShapeDtypeStruct((_S * _B, _H), jnp.float32),
        grid=(_S,),
        in_specs=[
            pl.BlockSpec((_S * _B, 3 * _H), const),
            pl.BlockSpec((1, _B, 1), lambda s: (s, 0, 0)),
            pl.BlockSpec((_H, 3 * _H), const),
            pl.BlockSpec((1, 3 * _H), const),
            pl.BlockSpec((2 * _H, _H), const),
            pl.BlockSpec((4, _H), const),
            pl.BlockSpec((1, _H), const),
            pl.BlockSpec((_H, _H), const),
            pl.BlockSpec((1, _H), const),
            pl.BlockSpec((_H, 4 * _C + 2), const),
            pl.BlockSpec((1, 4 * _C + 2), const),
            pl.BlockSpec((_H, _C * _H), const),
            pl.BlockSpec((1, _C * _H), const),
            pl.BlockSpec((_H, _H), const),
            pl.BlockSpec((1, _H), const),
            pl.BlockSpec((3 * _H, _H), const),
            pl.BlockSpec((4, _H), const),
            pl.BlockSpec((1, _H), const),
        ],
        out_specs=pl.BlockSpec((_B, _H), lambda s: (s, 0)),
        scratch_shapes=[
            pltpu.VMEM((_BC, _H), jnp.float32),
            pltpu.VMEM((_B, _C), jnp.float32),
            pltpu.VMEM((_B, _C), jnp.float32),
            pltpu.VMEM((_B, _H), jnp.float32),
            pltpu.VMEM((_BC, _H), jnp.float32),
        ],
        compiler_params=pltpu.CompilerParams(
            dimension_semantics=("arbitrary",),
            vmem_limit_bytes=50 * 1024 * 1024,
        ),
        name="cfrm_scan",
        interpret=interpret,
    )(gi_all, maskS, whhT.astype(bf), bhh2, w1lc.astype(bf), w1s, b1_2,
      ctrl_w2.astype(bf), b2_2, hw.astype(bf), hb, cc_w.astype(bf), cc_b2,
      att_w.astype(bf), attb2, o1m.astype(bf), o1s, ob1)

    h1_bs = jnp.swapaxes(h1_sb.reshape(_S, _B, _H), 0, 1).reshape(_B * _S, _H)
    logits = pl.pallas_call(
        _logits_kernel,
        out_shape=jax.ShapeDtypeStruct((_B * _S, _V), jnp.float32),
        grid=(_V // _NT,),
        in_specs=[
            pl.BlockSpec((_B * _S, _H), lambda j: (0, 0)),
            pl.BlockSpec((_H, _NT), lambda j: (0, j)),
            pl.BlockSpec((1, _NT), lambda j: (0, j)),
        ],
        out_specs=pl.BlockSpec((_B * _S, _NT), lambda j: (0, j)),
        compiler_params=pltpu.CompilerParams(
            dimension_semantics=("parallel",),
        ),
        name="cfrm_logits",
        interpret=interpret,
    )(h1_bs.astype(bf), out_w2.astype(bf), ob2)
    return logits.reshape(_B, _S, _V)


def kernel(tokens, emb, gru_wih, gru_whh, gru_bih, gru_bhh, ctrl_w1, ctrl_b1,
           ctrl_w2, ctrl_b2, gate_w, gate_b, assign_w, assign_b, nov_w, nov_b,
           relax_w, relax_b, cc_w, cc_b, cs_w, cs_b, md_w, md_b, att_w, att_b,
           out_w1, out_b1, out_w2, out_b2, *, interpret=False):
    x = emb[tokens]
    x_flat = jnp.swapaxes(x, 0, 1).reshape(_S * _B, _E)
    maskS = (tokens != 0).astype(jnp.float32).T[:, :, None]
    hw = jnp.concatenate([gate_w, assign_w, cs_w, md_w, nov_w, relax_w], axis=1)
    hb = jnp.concatenate([gate_b, assign_b, cs_b, md_b, nov_b, relax_b])[None, :]
    return _run(x_flat, maskS, gru_wih.T, gru_bih[None, :], gru_whh.T,
                gru_bhh[None, :], ctrl_w1[:2 * _H], ctrl_w1[2 * _H:],
                ctrl_b1[None, :], ctrl_w2, ctrl_b2[None, :], hw, hb,
                cc_w, cc_b[None, :], att_w, att_b[None, :],
                out_w1[:3 * _H], out_w1[3 * _H:], out_b1[None, :],
                out_w2, out_b2[None, :], interpret=interpret)


# speedup vs baseline: 12.0918x; 1.0702x over previous
import functools
import math

import jax
import jax.numpy as jnp
from jax import lax
from jax.experimental import pallas as pl
from jax.experimental.pallas import tpu as pltpu

_V, _C, _H, _E = 32000, 32, 256, 256
_B, _S = 8, 128
_EPS = 1e-4
_BC = _B * _C
_NEG = -1e30
_INVSQRT2 = 1.0 / math.sqrt(2.0)


def _dot(a, b):
    return jnp.dot(a, b, preferred_element_type=jnp.float32)


def _dg(a, b, dims):
    return lax.dot_general(a, b, (dims, ((), ())),
                           preferred_element_type=jnp.float32)


def _split(a):
    hi = a.astype(jnp.bfloat16).astype(jnp.float32)
    return hi, a - hi


def _xl(a, b):
    hi, lo = _split(a)
    return _dot(hi, b) + _dot(lo, b)


def _xr(a, b):
    hi, lo = _split(b)
    return _dot(a, hi) + _dot(a, lo)


def _consts():
    f32 = jnp.float32
    i32 = jnp.int32
    bmask8 = (lax.broadcasted_iota(i32, (_B, _BC), 1) % _B
              == lax.broadcasted_iota(i32, (_B, _BC), 0)).astype(f32)
    Q = (lax.broadcasted_iota(i32, (_C, _BC), 1) // _B
         == lax.broadcasted_iota(i32, (_C, _BC), 0)).astype(f32)
    P = (lax.broadcasted_iota(i32, (_BC, _C), 0) // _B
         == lax.broadcasted_iota(i32, (_BC, _C), 1)).astype(f32)
    eexp = (lax.broadcasted_iota(i32, (_BC, _B), 0) % _B
            == lax.broadcasted_iota(i32, (_BC, _B), 1)).astype(f32)
    eye = (lax.broadcasted_iota(i32, (_BC, _BC), 0)
           == lax.broadcasted_iota(i32, (_BC, _BC), 1)).astype(f32)
    same = (lax.broadcasted_iota(i32, (_BC, _BC), 0) % _B
            == lax.broadcasted_iota(i32, (_BC, _BC), 1) % _B)
    return bmask8, Q, P, eexp, eye, same


def _to_row(m, Q, bmask8):
    return jnp.sum(_xl(m, Q) * bmask8, axis=0, keepdims=True)


def _row_to_col(r, eye):
    return _dg(eye, _split(r)[0], ((1,), (1,))) + \
        _dg(eye, r - _split(r)[0], ((1,), (1,)))


def _col_to_row(c, eye):
    hi, lo = _split(c)
    return _dg(hi, eye, ((0,), (0,))) + _dg(lo, eye, ((0,), (0,)))


def _row_to_m(r, P, bmask8):
    return _xl(jnp.broadcast_to(r, (_B, _BC)) * bmask8, P)


def _x2(a, b):
    ah, al = _split(a)
    bh, bl = _split(b)
    return _dot(ah, bh) + _dot(ah, bl) + _dot(al, bh)


def _to_col(m, eexp, P):
    return jnp.sum(_xr(eexp, m) * P, axis=-1, keepdims=True)


def _summarize(c2d, sp, ma, bmask8, Q, eexp, eye):
    prec = 1.0 / (sp + _EPS)
    scores = ma + jnp.log(prec + _EPS)
    smax = jnp.max(scores, axis=-1, keepdims=True)
    e = jnp.exp(scores - smax)
    alpha = e / jnp.sum(e, axis=-1, keepdims=True)
    A = _xl(alpha, Q) * bmask8
    core = _dot(A, c2d)
    unc = jnp.sum(alpha * sp, axis=-1, keepdims=True)
    core_exp = _xr(eexp, core)
    sqd = jnp.mean((c2d - core_exp) ** 2, axis=-1, keepdims=True)
    div = _x2(A, sqd)
    mmax = jnp.max(ma, axis=-1, keepdims=True)
    en = jnp.log(jnp.sum(jnp.exp(ma - mmax), axis=-1, keepdims=True)) + mmax
    ent = -jnp.sum(alpha * jnp.log(jnp.maximum(alpha, 1e-8)),
                   axis=-1, keepdims=True)
    return core, unc, div, en, ent, alpha


def _scan_kernel(gi_ref, mask_ref, whhT_ref, bhh_ref,
                 w1lc_ref, w1s_ref, b1_ref, w2_ref, b2_ref,
                 hw_ref, hb_ref, ccb_ref, attb_ref,
                 o1m_ref, o1s_ref, ob1_ref,
                 h1_ref,
                 c2d_ref, sp_ref, ma_ref, h_ref, cnd_ref, core_ref, scal_ref):
    s = pl.program_id(0)
    bmask8, Q, P, eexp, eye, same = _consts()

    @pl.when(s == 0)
    def _init():
        c2d_ref[...] = jnp.zeros_like(c2d_ref)
        sp_ref[...] = jnp.ones_like(sp_ref)
        ma_ref[...] = jnp.zeros_like(ma_ref)
        core_ref[...] = jnp.zeros_like(core_ref)
        lc = math.log(_C)
        col8 = lax.broadcasted_iota(jnp.int32, (_B, 8), 1)
        scal_ref[...] = jnp.where(
            col8 == 0, 1.0, jnp.where((col8 == 2) | (col8 == 3), lc, 0.0))
        gi0 = gi_ref[0:_B, :]
        gh0 = jnp.broadcast_to(bhh_ref[...], (_B, 3 * _H))
        r0 = jax.nn.sigmoid(gi0[:, :_H] + gh0[:, :_H])
        z0 = jax.nn.sigmoid(gi0[:, _H:2 * _H] + gh0[:, _H:2 * _H])
        n0 = jnp.tanh(gi0[:, 2 * _H:] + r0 * gh0[:, 2 * _H:])
        h_ref[...] = (1.0 - z0) * n0

    valid = mask_ref[0]
    local = h_ref[...]

    sn = jnp.minimum(s + 1, _S - 1)
    gi = gi_ref[pl.ds(pl.multiple_of(sn * _B, _B), _B), :]
    gh = _dot(local.astype(jnp.bfloat16), whhT_ref[...]) + bhh_ref[...]
    r = jax.nn.sigmoid(gi[:, :_H] + gh[:, :_H])
    z = jax.nn.sigmoid(gi[:, _H:2 * _H] + gh[:, _H:2 * _H])
    n = jnp.tanh(gi[:, 2 * _H:] + r * gh[:, 2 * _H:])
    h_ref[...] = (1.0 - z) * n + z * local

    c2d = c2d_ref[...]
    sp = sp_ref[...]
    ma = ma_ref[...]

    core = core_ref[...]
    unc = scal_ref[:, 0:1]
    div = scal_ref[:, 1:2]
    en = scal_ref[:, 2:3]
    ent = scal_ref[:, 3:4]
    pre = _dot(jnp.concatenate([local, core], axis=1).astype(jnp.bfloat16),
               w1lc_ref[...]) + b1_ref[...]
    pre = (pre + unc * w1s_ref[0:1, :] + div * w1s_ref[1:2, :]
           + en * w1s_ref[2:3, :] + ent * w1s_ref[3:4, :])
    ctrl = jnp.tanh(_dot(jnp.tanh(pre).astype(jnp.bfloat16), w2_ref[...])
                    + b2_ref[...])
    ctrl_bf = ctrl.astype(jnp.bfloat16)

    y = _dot(ctrl_bf, hw_ref[...])
    hs = y[:, 0:4 * _C + 2] + hb_ref[...]
    attr = y[:, _H:2 * _H] + attb_ref[...]
    cand = y[:, 2 * _H:] + ccb_ref[...]
    gate = jax.nn.sigmoid(hs[:, 0:_C]) * valid
    ae = hs[:, _C:2 * _C]
    ae = jnp.exp(ae - jnp.max(ae, axis=-1, keepdims=True))
    assign = ae / jnp.sum(ae, axis=-1, keepdims=True)
    cs_raw = hs[:, 2 * _C:3 * _C]
    cand_sp = (jnp.maximum(cs_raw, 0.0)
               + jnp.log(1.0 + jnp.exp(-jnp.abs(cs_raw))) + _EPS)
    mdel = jnp.tanh(hs[:, 3 * _C:4 * _C])
    nov = jax.nn.sigmoid(hs[:, 4 * _C:4 * _C + 1]) * valid
    relax = jax.nn.sigmoid(hs[:, 4 * _C + 1:4 * _C + 2]) * valid

    for c in range(_C):
        cnd_ref[c * _B:(c + 1) * _B, :] = cand[:, c * _H:(c + 1) * _H]
    cand2d = cnd_ref[...]

    strength = gate * assign
    st_col = _to_col(strength, eexp, P)
    c2d = c2d + st_col * (cand2d - c2d)
    sp = sp + strength * (cand_sp - sp)
    ma = ma + strength * mdel

    anr = _xr(eexp, jnp.concatenate([attr, nov, relax], axis=1))
    attr_exp = anr[:, :_H]
    nov_col = anr[:, _H:_H + 1]
    relax_col = anr[:, _H + 1:_H + 2]
    c2d = c2d + 0.1 * nov_col * (attr_exp - c2d)

    sp_row = _to_row(sp, Q, bmask8)
    ma_row = _to_row(ma, Q, bmask8)
    sp_col = _to_col(sp, eexp, P)
    sq_col = jnp.sum(c2d * c2d, axis=-1, keepdims=True)
    sq_row = _col_to_row(sq_col, eye)
    G = _dg(c2d, c2d, ((1,), (1,)))
    d2 = jnp.maximum(sq_col + sq_row - 2.0 * G, 0.0)
    scale = sp_col + sp_row + _EPS
    compat = jnp.where(same, -d2 / scale + ma_row, _NEG)
    cmax = jnp.max(compat, axis=-1, keepdims=True)
    cexp = jnp.exp(compat - cmax)
    mixing = cexp / jnp.sum(cexp, axis=-1, keepdims=True)
    mc = _dot(mixing, c2d)
    msp = jnp.sum(mixing * sp_row, axis=-1, keepdims=True)
    mma = jnp.sum(mixing * ma_row, axis=-1, keepdims=True)

    c2d = (1.0 - relax_col) * c2d + relax_col * mc
    sp = (1.0 - relax) * sp + relax * _row_to_m(_col_to_row(msp, eye), P, bmask8)
    ma = (1.0 - relax) * ma + relax * _row_to_m(_col_to_row(mma, eye), P, bmask8)

    c2d_ref[...] = c2d
    sp_ref[...] = sp
    ma_ref[...] = ma

    core2, unc2, div2, en2, ent2, alpha2 = _summarize(
        c2d, sp, ma, bmask8, Q, eexp, eye)
    core_ref[...] = core2
    scal_ref[...] = jnp.concatenate(
        [unc2, div2, en2, ent2, jnp.zeros((_B, 4), jnp.float32)], axis=1)
    cidx = lax.broadcasted_iota(jnp.int32, (_B, _C), 1)
    amax = jnp.max(alpha2, axis=-1, keepdims=True)
    idx = jnp.min(jnp.where(alpha2 == amax, cidx, _C), axis=-1, keepdims=True)
    jj = lax.broadcasted_iota(jnp.int32, (_B, _BC), 1)
    bb = lax.broadcasted_iota(jnp.int32, (_B, _BC), 0)
    oh = (((jj // _B) == idx) & ((jj % _B) == bb)).astype(jnp.float32)
    strongest = _xr(oh, c2d)

    feat = jnp.concatenate([local, core2, strongest], axis=1)
    h1 = _dot(feat.astype(jnp.bfloat16), o1m_ref[...]) + ob1_ref[...]
    h1 = (h1 + unc2 * o1s_ref[0:1, :] + div2 * o1s_ref[1:2, :]
          + en2 * o1s_ref[2:3, :] + ent2 * o1s_ref[3:4, :])
    h1 = 0.5 * h1 * (1.0 + lax.erf(h1 * _INVSQRT2))
    h1_ref[...] = h1


def _logits_kernel(a_ref, w_ref, b_ref, o_ref):
    o_ref[...] = _dot(a_ref[...], w_ref[...]) + b_ref[...]


_NT = 1280


@functools.partial(jax.jit, static_argnames=("interpret",))
def _run(x_flat, maskS, wihT, bih2, whhT, bhh2, w1lc, w1s, b1_2, ctrl_w2,
         b2_2, hw, hb, cc_w, cc_b2, att_w, attb2, o1m, o1s, ob1, out_w2,
         ob2, interpret=False):
    const = lambda s: (0, 0)
    bf = jnp.bfloat16
    gi_all = pl.pallas_call(
        _logits_kernel,
        out_shape=jax.ShapeDtypeStruct((_S * _B, 3 * _H), jnp.float32),
        name="cfrm_gi",
        interpret=interpret,
    )(x_flat.astype(bf), wihT.astype(bf), bih2)

    hw_wide = jnp.concatenate(
        [hw, jnp.zeros((_H, _H - (4 * _C + 2)), hw.dtype), att_w, cc_w],
        axis=1)
    h1_sb = pl.pallas_call(
        _scan_kernel,
        out_shape=jax.ShapeDtypeStruct((_S * _B, _H), jnp.float32),
        grid=(_S,),
        in_specs=[
            pl.BlockSpec((_S * _B, 3 * _H), const),
            pl.BlockSpec((1, _B, 1), lambda s: (s, 0, 0)),
            pl.BlockSpec((_H, 3 * _H), const),
            pl.BlockSpec((1, 3 * _H), const),
            pl.BlockSpec((2 * _H, _H), const),
            pl.BlockSpec((4, _H), const),
            pl.BlockSpec((1, _H), const),
            pl.BlockSpec((_H, _H), const),
            pl.BlockSpec((1, _H), const),
            pl.BlockSpec((_H, 2 * _H + _C * _H), const),
            pl.BlockSpec((1, 4 * _C + 2), const),
            pl.BlockSpec((1, _C * _H), const),
            pl.BlockSpec((1, _H), const),
            pl.BlockSpec((3 * _H, _H), const),
            pl.BlockSpec((4, _H), const),
            pl.BlockSpec((1, _H), const),
        ],
        out_specs=pl.BlockSpec((_B, _H), lambda s: (s, 0)),
        scratch_shapes=[
            pltpu.VMEM((_BC, _H), jnp.float32),
            pltpu.VMEM((_B, _C), jnp.float32),
            pltpu.VMEM((_B, _C), jnp.float32),
            pltpu.VMEM((_B, _H), jnp.float32),
            pltpu.VMEM((_BC, _H), jnp.float32),
            pltpu.VMEM((_B, _H), jnp.float32),
            pltpu.VMEM((_B, 8), jnp.float32),
        ],
        compiler_params=pltpu.CompilerParams(
            dimension_semantics=("arbitrary",),
            vmem_limit_bytes=50 * 1024 * 1024,
        ),
        name="cfrm_scan",
        interpret=interpret,
    )(gi_all, maskS, whhT.astype(bf), bhh2, w1lc.astype(bf), w1s, b1_2,
      ctrl_w2.astype(bf), b2_2, hw_wide.astype(bf), hb, cc_b2,
      attb2, o1m.astype(bf), o1s, ob1)

    h1_bs = jnp.swapaxes(h1_sb.reshape(_S, _B, _H), 0, 1).reshape(_B * _S, _H)
    logits = pl.pallas_call(
        _logits_kernel,
        out_shape=jax.ShapeDtypeStruct((_B * _S, _V), jnp.float32),
        grid=(_V // _NT,),
        in_specs=[
            pl.BlockSpec((_B * _S, _H), lambda j: (0, 0)),
            pl.BlockSpec((_H, _NT), lambda j: (0, j)),
            pl.BlockSpec((1, _NT), lambda j: (0, j)),
        ],
        out_specs=pl.BlockSpec((_B * _S, _NT), lambda j: (0, j)),
        compiler_params=pltpu.CompilerParams(
            dimension_semantics=("parallel",),
        ),
        name="cfrm_logits",
        interpret=interpret,
    )(h1_bs, out_w2, ob2)
    return logits.reshape(_B, _S, _V)


def kernel(tokens, emb, gru_wih, gru_whh, gru_bih, gru_bhh, ctrl_w1, ctrl_b1,
           ctrl_w2, ctrl_b2, gate_w, gate_b, assign_w, assign_b, nov_w, nov_b,
           relax_w, relax_b, cc_w, cc_b, cs_w, cs_b, md_w, md_b, att_w, att_b,
           out_w1, out_b1, out_w2, out_b2, *, interpret=False):
    x = emb[tokens]
    x_flat = jnp.swapaxes(x, 0, 1).reshape(_S * _B, _E)
    maskS = (tokens != 0).astype(jnp.float32).T[:, :, None]
    hw = jnp.concatenate([gate_w, assign_w, cs_w, md_w, nov_w, relax_w], axis=1)
    hb = jnp.concatenate([gate_b, assign_b, cs_b, md_b, nov_b, relax_b])[None, :]
    return _run(x_flat, maskS, gru_wih.T, gru_bih[None, :], gru_whh.T,
                gru_bhh[None, :], ctrl_w1[:2 * _H], ctrl_w1[2 * _H:],
                ctrl_b1[None, :], ctrl_w2, ctrl_b2[None, :], hw, hb,
                cc_w, cc_b[None, :], att_w, att_b[None, :],
                out_w1[:3 * _H], out_w1[3 * _H:], out_b1[None, :],
                out_w2, out_b2[None, :], interpret=interpret)


# speedup vs baseline: 12.3877x; 1.0245x over previous
import functools
import math

import jax
import jax.numpy as jnp
from jax import lax
from jax.experimental import pallas as pl
from jax.experimental.pallas import tpu as pltpu

_V, _C, _H, _E = 32000, 32, 256, 256
_B, _S = 8, 128
_EPS = 1e-4
_BC = _B * _C
_NEG = -1e30
_INVSQRT2 = 1.0 / math.sqrt(2.0)


def _dot(a, b):
    return jnp.dot(a, b, preferred_element_type=jnp.float32)


def _dg(a, b, dims):
    return lax.dot_general(a, b, (dims, ((), ())),
                           preferred_element_type=jnp.float32)


def _split(a):
    hi = a.astype(jnp.bfloat16).astype(jnp.float32)
    return hi, a - hi


def _xl(a, b):
    hi, lo = _split(a)
    return _dot(hi, b) + _dot(lo, b)


def _xr(a, b):
    hi, lo = _split(b)
    return _dot(a, hi) + _dot(a, lo)


def _consts():
    f32 = jnp.float32
    i32 = jnp.int32
    bmask8 = (lax.broadcasted_iota(i32, (_B, _BC), 1) % _B
              == lax.broadcasted_iota(i32, (_B, _BC), 0)).astype(f32)
    Q = (lax.broadcasted_iota(i32, (_C, _BC), 1) // _B
         == lax.broadcasted_iota(i32, (_C, _BC), 0)).astype(f32)
    P = (lax.broadcasted_iota(i32, (_BC, _C), 0) // _B
         == lax.broadcasted_iota(i32, (_BC, _C), 1)).astype(f32)
    eexp = (lax.broadcasted_iota(i32, (_BC, _B), 0) % _B
            == lax.broadcasted_iota(i32, (_BC, _B), 1)).astype(f32)
    eye = (lax.broadcasted_iota(i32, (_BC, _BC), 0)
           == lax.broadcasted_iota(i32, (_BC, _BC), 1)).astype(f32)
    same = (lax.broadcasted_iota(i32, (_BC, _BC), 0) % _B
            == lax.broadcasted_iota(i32, (_BC, _BC), 1) % _B)
    return bmask8, Q, P, eexp, eye, same


def _to_row(m, Q, bmask8):
    return jnp.sum(_xl(m, Q) * bmask8, axis=0, keepdims=True)


def _row_to_col(r, eye):
    return _dg(eye, _split(r)[0], ((1,), (1,))) + \
        _dg(eye, r - _split(r)[0], ((1,), (1,)))


def _col_to_row(c, eye):
    hi, lo = _split(c)
    return _dg(hi, eye, ((0,), (0,))) + _dg(lo, eye, ((0,), (0,)))


def _row_to_m(r, P, bmask8):
    return _xl(jnp.broadcast_to(r, (_B, _BC)) * bmask8, P)


def _x2(a, b):
    ah, al = _split(a)
    bh, bl = _split(b)
    return _dot(ah, bh) + _dot(ah, bl) + _dot(al, bh)


def _to_col(m, eexp, P):
    return jnp.sum(_xr(eexp, m) * P, axis=-1, keepdims=True)


def _summarize(c2d, sp, ma, bmask8, Q, eexp, eye):
    prec = 1.0 / (sp + _EPS)
    scores = ma + jnp.log(prec + _EPS)
    smax = jnp.max(scores, axis=-1, keepdims=True)
    e = jnp.exp(scores - smax)
    alpha = e / jnp.sum(e, axis=-1, keepdims=True)
    A = _xl(alpha, Q) * bmask8
    core = _dot(A, c2d)
    unc = jnp.sum(alpha * sp, axis=-1, keepdims=True)
    core_exp = _xr(eexp, core)
    sqd = jnp.mean((c2d - core_exp) ** 2, axis=-1, keepdims=True)
    div = _x2(A, sqd)
    mmax = jnp.max(ma, axis=-1, keepdims=True)
    en = jnp.log(jnp.sum(jnp.exp(ma - mmax), axis=-1, keepdims=True)) + mmax
    ent = -jnp.sum(alpha * jnp.log(jnp.maximum(alpha, 1e-8)),
                   axis=-1, keepdims=True)
    return core, unc, div, en, ent, alpha


def _scan_kernel(gi_ref, mask_ref, whhT_ref, bhh_ref,
                 w1lc_ref, w1s_ref, b1_ref, w2_ref, b2_ref,
                 hw_ref, hb_ref, ccb_ref, attb_ref,
                 o1m_ref, o1s_ref, ob1_ref,
                 h1_ref,
                 c2d_ref, sp_ref, ma_ref, h_ref, cnd_ref, core_ref, scal_ref,
                 alpha_ref, lp_ref):
    s = pl.program_id(0)
    bmask8, Q, P, eexp, eye, same = _consts()

    @pl.when(s == 0)
    def _init():
        c2d_ref[...] = jnp.zeros_like(c2d_ref)
        sp_ref[...] = jnp.ones_like(sp_ref)
        ma_ref[...] = jnp.zeros_like(ma_ref)
        core_ref[...] = jnp.zeros_like(core_ref)
        lc = math.log(_C)
        col8 = lax.broadcasted_iota(jnp.int32, (_B, 8), 1)
        scal_ref[...] = jnp.where(
            col8 == 0, 1.0, jnp.where((col8 == 2) | (col8 == 3), lc, 0.0))
        alpha_ref[...] = jnp.full((_B, _C), 1.0 / _C, jnp.float32)
        lp_ref[...] = jnp.zeros_like(lp_ref)
        gi0 = gi_ref[0:_B, :]
        gh0 = jnp.broadcast_to(bhh_ref[...], (_B, 3 * _H))
        r0 = jax.nn.sigmoid(gi0[:, :_H] + gh0[:, :_H])
        z0 = jax.nn.sigmoid(gi0[:, _H:2 * _H] + gh0[:, _H:2 * _H])
        n0 = jnp.tanh(gi0[:, 2 * _H:] + r0 * gh0[:, 2 * _H:])
        h_ref[...] = (1.0 - z0) * n0

    valid = mask_ref[0]
    local = h_ref[...]
    c2d = c2d_ref[...]
    sp = sp_ref[...]
    ma = ma_ref[...]
    core = core_ref[...]
    unc = scal_ref[:, 0:1]
    div = scal_ref[:, 1:2]
    en = scal_ref[:, 2:3]
    ent = scal_ref[:, 3:4]

    c2d0 = c2d

    pre = _dot(jnp.concatenate([local, core], axis=1).astype(jnp.bfloat16),
               w1lc_ref[...]) + b1_ref[...]
    pre = (pre + unc * w1s_ref[0:1, :] + div * w1s_ref[1:2, :]
           + en * w1s_ref[2:3, :] + ent * w1s_ref[3:4, :])
    ctrl = jnp.tanh(_dot(jnp.tanh(pre).astype(jnp.bfloat16), w2_ref[...])
                    + b2_ref[...])
    ctrl_bf = ctrl.astype(jnp.bfloat16)

    y = _dot(ctrl_bf, hw_ref[...])
    hs = y[:, 0:4 * _C + 2] + hb_ref[...]
    attr = y[:, _H:2 * _H] + attb_ref[...]
    cand = y[:, 2 * _H:] + ccb_ref[...]
    gate = jax.nn.sigmoid(hs[:, 0:_C]) * valid
    ae = hs[:, _C:2 * _C]
    ae = jnp.exp(ae - jnp.max(ae, axis=-1, keepdims=True))
    assign = ae / jnp.sum(ae, axis=-1, keepdims=True)
    cs_raw = hs[:, 2 * _C:3 * _C]
    cand_sp = (jnp.maximum(cs_raw, 0.0)
               + jnp.log(1.0 + jnp.exp(-jnp.abs(cs_raw))) + _EPS)
    mdel = jnp.tanh(hs[:, 3 * _C:4 * _C])
    nov = jax.nn.sigmoid(hs[:, 4 * _C:4 * _C + 1]) * valid
    relax = jax.nn.sigmoid(hs[:, 4 * _C + 1:4 * _C + 2]) * valid

    for c in range(_C):
        cnd_ref[c * _B:(c + 1) * _B, :] = cand[:, c * _H:(c + 1) * _H]
    cand2d = cnd_ref[...]

    strength = gate * assign
    st_col = _to_col(strength, eexp, P)
    c2d = c2d + st_col * (cand2d - c2d)
    sp = sp + strength * (cand_sp - sp)
    ma = ma + strength * mdel

    anr = _xr(eexp, jnp.concatenate([attr, nov, relax], axis=1))
    attr_exp = anr[:, :_H]
    nov_col = anr[:, _H:_H + 1]
    relax_col = anr[:, _H + 1:_H + 2]
    c2d = c2d + 0.1 * nov_col * (attr_exp - c2d)

    sp_row = _to_row(sp, Q, bmask8)
    ma_row = _to_row(ma, Q, bmask8)
    sp_col = _to_col(sp, eexp, P)
    sq_col = jnp.sum(c2d * c2d, axis=-1, keepdims=True)
    sq_row = _col_to_row(sq_col, eye)
    G = _dg(c2d, c2d, ((1,), (1,)))
    d2 = jnp.maximum(sq_col + sq_row - 2.0 * G, 0.0)
    scale = sp_col + sp_row + _EPS
    compat = jnp.where(same, -d2 / scale + ma_row, _NEG)
    alpha2 = alpha_ref[...]
    cidx = lax.broadcasted_iota(jnp.int32, (_B, _C), 1)
    amax = jnp.max(alpha2, axis=-1, keepdims=True)
    idx = jnp.min(jnp.where(alpha2 == amax, cidx, _C), axis=-1, keepdims=True)
    jj = lax.broadcasted_iota(jnp.int32, (_B, _BC), 1)
    bb = lax.broadcasted_iota(jnp.int32, (_B, _BC), 0)
    oh = (((jj // _B) == idx) & ((jj % _B) == bb)).astype(jnp.float32)
    strongest = _xr(oh, c2d0)
    feat = jnp.concatenate([lp_ref[...], core, strongest], axis=1)
    h1 = _dot(feat.astype(jnp.bfloat16), o1m_ref[...]) + ob1_ref[...]
    h1 = (h1 + unc * o1s_ref[0:1, :] + div * o1s_ref[1:2, :]
          + en * o1s_ref[2:3, :] + ent * o1s_ref[3:4, :])
    h1 = 0.5 * h1 * (1.0 + lax.erf(h1 * _INVSQRT2))
    h1_ref[...] = h1

    cmax = jnp.max(compat, axis=-1, keepdims=True)
    cexp = jnp.exp(compat - cmax)
    mixing = cexp / jnp.sum(cexp, axis=-1, keepdims=True)
    mc = _dot(mixing, c2d)
    msp = jnp.sum(mixing * sp_row, axis=-1, keepdims=True)
    mma = jnp.sum(mixing * ma_row, axis=-1, keepdims=True)

    c2d = (1.0 - relax_col) * c2d + relax_col * mc
    sp = (1.0 - relax) * sp + relax * _row_to_m(_col_to_row(msp, eye), P, bmask8)
    ma = (1.0 - relax) * ma + relax * _row_to_m(_col_to_row(mma, eye), P, bmask8)

    c2d_ref[...] = c2d
    sp_ref[...] = sp
    ma_ref[...] = ma
    lp_ref[...] = local

    sn = jnp.minimum(s + 1, _S - 1)
    gi = gi_ref[pl.ds(pl.multiple_of(sn * _B, _B), _B), :]
    gh = _dot(local.astype(jnp.bfloat16), whhT_ref[...]) + bhh_ref[...]
    r = jax.nn.sigmoid(gi[:, :_H] + gh[:, :_H])
    z = jax.nn.sigmoid(gi[:, _H:2 * _H] + gh[:, _H:2 * _H])
    n = jnp.tanh(gi[:, 2 * _H:] + r * gh[:, 2 * _H:])
    h_ref[...] = (1.0 - z) * n + z * local


    core2, unc2, div2, en2, ent2, alpha2 = _summarize(
        c2d, sp, ma, bmask8, Q, eexp, eye)
    core_ref[...] = core2
    scal_ref[...] = jnp.concatenate(
        [unc2, div2, en2, ent2, jnp.zeros((_B, 4), jnp.float32)], axis=1)
    alpha_ref[...] = alpha2


def _logits_kernel(a_ref, w_ref, b_ref, o_ref):
    o_ref[...] = _dot(a_ref[...], w_ref[...]) + b_ref[...]


_NT = 3200


@functools.partial(jax.jit, static_argnames=("interpret",))
def _run(x_flat, maskS, wihT, bih2, whhT, bhh2, w1lc, w1s, b1_2, ctrl_w2,
         b2_2, hw, hb, cc_w, cc_b2, att_w, attb2, o1m, o1s, ob1, out_w2,
         ob2, interpret=False):
    const = lambda s: (0, 0)
    bf = jnp.bfloat16
    gi_all = pl.pallas_call(
        _logits_kernel,
        out_shape=jax.ShapeDtypeStruct((_S * _B, 3 * _H), jnp.float32),
        name="cfrm_gi",
        interpret=interpret,
    )(x_flat.astype(bf), wihT.astype(bf), bih2)

    hw_wide = jnp.concatenate(
        [hw, jnp.zeros((_H, _H - (4 * _C + 2)), hw.dtype), att_w, cc_w],
        axis=1)
    h1_sb = pl.pallas_call(
        _scan_kernel,
        out_shape=jax.ShapeDtypeStruct((_S * _B, _H), jnp.float32),
        grid=(_S + 1,),
        in_specs=[
            pl.BlockSpec((_S * _B, 3 * _H), const),
            pl.BlockSpec((1, _B, 1),
                         lambda s: (jnp.minimum(s, _S - 1), 0, 0)),
            pl.BlockSpec((_H, 3 * _H), const),
            pl.BlockSpec((1, 3 * _H), const),
            pl.BlockSpec((2 * _H, _H), const),
            pl.BlockSpec((4, _H), const),
            pl.BlockSpec((1, _H), const),
            pl.BlockSpec((_H, _H), const),
            pl.BlockSpec((1, _H), const),
            pl.BlockSpec((_H, 2 * _H + _C * _H), const),
            pl.BlockSpec((1, 4 * _C + 2), const),
            pl.BlockSpec((1, _C * _H), const),
            pl.BlockSpec((1, _H), const),
            pl.BlockSpec((3 * _H, _H), const),
            pl.BlockSpec((4, _H), const),
            pl.BlockSpec((1, _H), const),
        ],
        out_specs=pl.BlockSpec((_B, _H),
                               lambda s: (jnp.maximum(s - 1, 0), 0)),
        scratch_shapes=[
            pltpu.VMEM((_BC, _H), jnp.float32),
            pltpu.VMEM((_B, _C), jnp.float32),
            pltpu.VMEM((_B, _C), jnp.float32),
            pltpu.VMEM((_B, _H), jnp.float32),
            pltpu.VMEM((_BC, _H), jnp.float32),
            pltpu.VMEM((_B, _H), jnp.float32),
            pltpu.VMEM((_B, 8), jnp.float32),
            pltpu.VMEM((_B, _C), jnp.float32),
            pltpu.VMEM((_B, _H), jnp.float32),
        ],
        compiler_params=pltpu.CompilerParams(
            dimension_semantics=("arbitrary",),
            vmem_limit_bytes=50 * 1024 * 1024,
        ),
        name="cfrm_scan",
        interpret=interpret,
    )(gi_all, maskS, whhT.astype(bf), bhh2, w1lc.astype(bf), w1s, b1_2,
      ctrl_w2.astype(bf), b2_2, hw_wide.astype(bf), hb, cc_b2,
      attb2, o1m.astype(bf), o1s, ob1)

    h1_bs = jnp.swapaxes(h1_sb.reshape(_S, _B, _H), 0, 1).reshape(_B * _S, _H)
    logits = pl.pallas_call(
        _logits_kernel,
        out_shape=jax.ShapeDtypeStruct((_B * _S, _V), jnp.float32),
        grid=(_V // _NT,),
        in_specs=[
            pl.BlockSpec((_B * _S, _H), lambda j: (0, 0)),
            pl.BlockSpec((_H, _NT), lambda j: (0, j)),
            pl.BlockSpec((1, _NT), lambda j: (0, j)),
        ],
        out_specs=pl.BlockSpec((_B * _S, _NT), lambda j: (0, j)),
        compiler_params=pltpu.CompilerParams(
            dimension_semantics=("parallel",),
            vmem_limit_bytes=48 * 1024 * 1024,
        ),
        name="cfrm_logits",
        interpret=interpret,
    )(h1_bs.astype(bf), out_w2.astype(bf), ob2)
    return logits.reshape(_B, _S, _V)


def kernel(tokens, emb, gru_wih, gru_whh, gru_bih, gru_bhh, ctrl_w1, ctrl_b1,
           ctrl_w2, ctrl_b2, gate_w, gate_b, assign_w, assign_b, nov_w, nov_b,
           relax_w, relax_b, cc_w, cc_b, cs_w, cs_b, md_w, md_b, att_w, att_b,
           out_w1, out_b1, out_w2, out_b2, *, interpret=False):
    x = emb[tokens]
    x_flat = jnp.swapaxes(x, 0, 1).reshape(_S * _B, _E)
    maskS = (tokens != 0).astype(jnp.float32).T[:, :, None]
    hw = jnp.concatenate([gate_w, assign_w, cs_w, md_w, nov_w, relax_w], axis=1)
    hb = jnp.concatenate([gate_b, assign_b, cs_b, md_b, nov_b, relax_b])[None, :]
    return _run(x_flat, maskS, gru_wih.T, gru_bih[None, :], gru_whh.T,
                gru_bhh[None, :], ctrl_w1[:2 * _H], ctrl_w1[2 * _H:],
                ctrl_b1[None, :], ctrl_w2, ctrl_b2[None, :], hw, hb,
                cc_w, cc_b[None, :], att_w, att_b[None, :],
                out_w1[:3 * _H], out_w1[3 * _H:], out_b1[None, :],
                out_w2, out_b2[None, :], interpret=interpret)


# speedup vs baseline: 13.2394x; 1.0688x over previous
import functools
import math

import jax
import jax.numpy as jnp
from jax import lax
from jax.experimental import pallas as pl
from jax.experimental.pallas import tpu as pltpu

_V, _C, _H, _E = 32000, 32, 256, 256
_B, _S = 8, 128
_EPS = 1e-4
_BC = _B * _C
_NEG = -1e30
_INVSQRT2 = 1.0 / math.sqrt(2.0)


def _dot(a, b):
    return jnp.dot(a, b, preferred_element_type=jnp.float32)


def _dg(a, b, dims):
    return lax.dot_general(a, b, (dims, ((), ())),
                           preferred_element_type=jnp.float32)


def _split(a):
    hi = a.astype(jnp.bfloat16).astype(jnp.float32)
    return hi, a - hi


def _xl(a, b):
    hi, lo = _split(a)
    return _dot(hi, b) + _dot(lo, b)


def _xr(a, b):
    hi, lo = _split(b)
    return _dot(a, hi) + _dot(a, lo)


def _consts():
    f32 = jnp.float32
    i32 = jnp.int32
    bmask8 = (lax.broadcasted_iota(i32, (_B, _BC), 1) % _B
              == lax.broadcasted_iota(i32, (_B, _BC), 0)).astype(f32)
    Q = (lax.broadcasted_iota(i32, (_C, _BC), 1) // _B
         == lax.broadcasted_iota(i32, (_C, _BC), 0)).astype(f32)
    P = (lax.broadcasted_iota(i32, (_BC, _C), 0) // _B
         == lax.broadcasted_iota(i32, (_BC, _C), 1)).astype(f32)
    eexp = (lax.broadcasted_iota(i32, (_BC, _B), 0) % _B
            == lax.broadcasted_iota(i32, (_BC, _B), 1)).astype(f32)
    eye = (lax.broadcasted_iota(i32, (_BC, _BC), 0)
           == lax.broadcasted_iota(i32, (_BC, _BC), 1)).astype(f32)
    same = (lax.broadcasted_iota(i32, (_BC, _BC), 0) % _B
            == lax.broadcasted_iota(i32, (_BC, _BC), 1) % _B)
    return bmask8, Q, P, eexp, eye, same


def _to_row(m, Q, bmask8):
    return jnp.sum(_xl(m, Q) * bmask8, axis=0, keepdims=True)


def _row_to_col(r, eye):
    return _dg(eye, _split(r)[0], ((1,), (1,))) + \
        _dg(eye, r - _split(r)[0], ((1,), (1,)))


def _col_to_row(c, eye):
    hi, lo = _split(c)
    return _dg(hi, eye, ((0,), (0,))) + _dg(lo, eye, ((0,), (0,)))


def _row_to_m(r, P, bmask8):
    return _xl(jnp.broadcast_to(r, (_B, _BC)) * bmask8, P)


def _x2(a, b):
    ah, al = _split(a)
    bh, bl = _split(b)
    return _dot(ah, bh) + _dot(ah, bl) + _dot(al, bh)


def _to_col(m, eexp, P):
    return jnp.sum(_xr(eexp, m) * P, axis=-1, keepdims=True)


def _summarize(c2d, sp, ma, bmask8, Q, eexp, eye):
    prec = 1.0 / (sp + _EPS)
    scores = ma + jnp.log(prec + _EPS)
    smax = jnp.max(scores, axis=-1, keepdims=True)
    e = jnp.exp(scores - smax)
    alpha = e / jnp.sum(e, axis=-1, keepdims=True)
    A = _xl(alpha, Q) * bmask8
    core = _dot(A, c2d)
    unc = jnp.sum(alpha * sp, axis=-1, keepdims=True)
    core_exp = _xr(eexp, core)
    sqd = jnp.mean((c2d - core_exp) ** 2, axis=-1, keepdims=True)
    div = _x2(A, sqd)
    mmax = jnp.max(ma, axis=-1, keepdims=True)
    en = jnp.log(jnp.sum(jnp.exp(ma - mmax), axis=-1, keepdims=True)) + mmax
    ent = -jnp.sum(alpha * jnp.log(jnp.maximum(alpha, 1e-8)),
                   axis=-1, keepdims=True)
    return core, unc, div, en, ent, alpha


def _scan_kernel(gi_ref, mask_ref, whhT_ref, bhh_ref,
                 w1lc_ref, w1s_ref, b1_ref, w2_ref, b2_ref,
                 hw_ref, hb_ref, ccb_ref, attb_ref,
                 o1m_ref, o1s_ref, ob1_ref,
                 h1_ref,
                 c2d_ref, sp_ref, ma_ref, h_ref, cnd_ref, core_ref, scal_ref,
                 alpha_ref, lp_ref):
    s = pl.program_id(0)
    bmask8, Q, P, eexp, eye, same = _consts()

    @pl.when(s == 0)
    def _init():
        c2d_ref[...] = jnp.zeros_like(c2d_ref)
        sp_ref[...] = jnp.ones_like(sp_ref)
        ma_ref[...] = jnp.zeros_like(ma_ref)
        core_ref[...] = jnp.zeros_like(core_ref)
        lc = math.log(_C)
        col8 = lax.broadcasted_iota(jnp.int32, (_B, 8), 1)
        scal_ref[...] = jnp.where(
            col8 == 0, 1.0, jnp.where((col8 == 2) | (col8 == 3), lc, 0.0))
        alpha_ref[...] = jnp.full((_B, _C), 1.0 / _C, jnp.float32)
        lp_ref[...] = jnp.zeros_like(lp_ref)
        gi0 = gi_ref[0:_B, :]
        gh0 = jnp.broadcast_to(bhh_ref[...], (_B, 3 * _H))
        r0 = jax.nn.sigmoid(gi0[:, :_H] + gh0[:, :_H])
        z0 = jax.nn.sigmoid(gi0[:, _H:2 * _H] + gh0[:, _H:2 * _H])
        n0 = jnp.tanh(gi0[:, 2 * _H:] + r0 * gh0[:, 2 * _H:])
        h_ref[...] = (1.0 - z0) * n0

    valid = mask_ref[0]
    local = h_ref[...]
    c2d = c2d_ref[...]
    sp = sp_ref[...]
    ma = ma_ref[...]
    core = core_ref[...]
    unc = scal_ref[:, 0:1]
    div = scal_ref[:, 1:2]
    en = scal_ref[:, 2:3]
    ent = scal_ref[:, 3:4]

    c2d0 = c2d

    pre = _dot(jnp.concatenate([local, core], axis=1).astype(jnp.bfloat16),
               w1lc_ref[...]) + b1_ref[...]
    pre = (pre + unc * w1s_ref[0:1, :] + div * w1s_ref[1:2, :]
           + en * w1s_ref[2:3, :] + ent * w1s_ref[3:4, :])
    ctrl = jnp.tanh(_dot(jnp.tanh(pre).astype(jnp.bfloat16), w2_ref[...])
                    + b2_ref[...])
    ctrl_bf = ctrl.astype(jnp.bfloat16)

    y = _dot(ctrl_bf, hw_ref[...])
    hs = y[:, 0:4 * _C + 2] + hb_ref[...]
    attr = y[:, _H:2 * _H] + attb_ref[...]
    cand = y[:, 2 * _H:] + ccb_ref[...]
    gate = jax.nn.sigmoid(hs[:, 0:_C]) * valid
    ae = hs[:, _C:2 * _C]
    ae = jnp.exp(ae - jnp.max(ae, axis=-1, keepdims=True))
    assign = ae / jnp.sum(ae, axis=-1, keepdims=True)
    cs_raw = hs[:, 2 * _C:3 * _C]
    cand_sp = (jnp.maximum(cs_raw, 0.0)
               + jnp.log(1.0 + jnp.exp(-jnp.abs(cs_raw))) + _EPS)
    mdel = jnp.tanh(hs[:, 3 * _C:4 * _C])
    nov = jax.nn.sigmoid(hs[:, 4 * _C:4 * _C + 1]) * valid
    relax = jax.nn.sigmoid(hs[:, 4 * _C + 1:4 * _C + 2]) * valid

    for c in range(_C):
        cnd_ref[c * _B:(c + 1) * _B, :] = cand[:, c * _H:(c + 1) * _H]
    cand2d = cnd_ref[...]

    strength = gate * assign
    st_col = _to_col(strength, eexp, P)
    c2d = c2d + st_col * (cand2d - c2d)
    sp = sp + strength * (cand_sp - sp)
    ma = ma + strength * mdel

    anr = _xr(eexp, jnp.concatenate([attr, nov, relax], axis=1))
    attr_exp = anr[:, :_H]
    nov_col = anr[:, _H:_H + 1]
    relax_col = anr[:, _H + 1:_H + 2]
    c2d = c2d + 0.1 * nov_col * (attr_exp - c2d)

    sp_row = _to_row(sp, Q, bmask8)
    ma_row = _to_row(ma, Q, bmask8)
    sp_col = _to_col(sp, eexp, P)
    sq_col = jnp.sum(c2d * c2d, axis=-1, keepdims=True)
    G = _dg(c2d, c2d, ((1,), (1,)))
    sq_row = jnp.sum(G * eye, axis=0, keepdims=True)
    d2 = jnp.maximum(sq_col + sq_row - 2.0 * G, 0.0)
    scale = sp_col + sp_row + _EPS
    compat = jnp.where(same, -d2 / scale + ma_row, _NEG)
    alpha2 = alpha_ref[...]
    cidx = lax.broadcasted_iota(jnp.int32, (_B, _C), 1)
    amax = jnp.max(alpha2, axis=-1, keepdims=True)
    idx = jnp.min(jnp.where(alpha2 == amax, cidx, _C), axis=-1, keepdims=True)
    jj = lax.broadcasted_iota(jnp.int32, (_B, _BC), 1)
    bb = lax.broadcasted_iota(jnp.int32, (_B, _BC), 0)
    oh = (((jj // _B) == idx) & ((jj % _B) == bb)).astype(jnp.float32)
    strongest = _dot(oh, c2d0)
    feat = jnp.concatenate([lp_ref[...], core, strongest], axis=1)
    h1 = _dot(feat.astype(jnp.bfloat16), o1m_ref[...]) + ob1_ref[...]
    h1 = (h1 + unc * o1s_ref[0:1, :] + div * o1s_ref[1:2, :]
          + en * o1s_ref[2:3, :] + ent * o1s_ref[3:4, :])
    h1 = 0.5 * h1 * (1.0 + lax.erf(h1 * _INVSQRT2))
    h1_ref[...] = h1

    cmax = jnp.max(compat, axis=-1, keepdims=True)
    cexp = jnp.exp(compat - cmax)
    mixing = cexp / jnp.sum(cexp, axis=-1, keepdims=True)
    mc = _dot(mixing, c2d)
    msp = jnp.sum(mixing * sp_row, axis=-1, keepdims=True)
    mma = jnp.sum(mixing * ma_row, axis=-1, keepdims=True)

    c2d = (1.0 - relax_col) * c2d + relax_col * mc
    mrows = _col_to_row(jnp.concatenate([msp, mma], axis=1), eye)
    sp = (1.0 - relax) * sp + relax * _row_to_m(mrows[0:1], P, bmask8)
    ma = (1.0 - relax) * ma + relax * _row_to_m(mrows[1:2], P, bmask8)

    c2d_ref[...] = c2d
    sp_ref[...] = sp
    ma_ref[...] = ma
    lp_ref[...] = local

    sn = jnp.minimum(s + 1, _S - 1)
    gi = gi_ref[pl.ds(pl.multiple_of(sn * _B, _B), _B), :]
    gh = _dg(local.astype(jnp.bfloat16), whhT_ref[...], ((1,), (1,))) + bhh_ref[...]
    r = jax.nn.sigmoid(gi[:, :_H] + gh[:, :_H])
    z = jax.nn.sigmoid(gi[:, _H:2 * _H] + gh[:, _H:2 * _H])
    n = jnp.tanh(gi[:, 2 * _H:] + r * gh[:, 2 * _H:])
    h_ref[...] = (1.0 - z) * n + z * local


    core2, unc2, div2, en2, ent2, alpha2 = _summarize(
        c2d, sp, ma, bmask8, Q, eexp, eye)
    core_ref[...] = core2
    scal_ref[...] = jnp.concatenate(
        [unc2, div2, en2, ent2, jnp.zeros((_B, 4), jnp.float32)], axis=1)
    alpha_ref[...] = alpha2


def _logits_kernel(a_ref, w_ref, b_ref, o_ref):
    o_ref[...] = _dot(a_ref[...], w_ref[...]) + b_ref[...]


def _gi_kernel(x_ref, wih_ref, b_ref, hw_ref, att_ref, cc_ref, whh_ref,
               w1_ref, w2_ref, o1_ref,
               gi_ref, wide_ref, whh_o, w1_o, w2_o, o1_o):
    gi_ref[...] = _dg(x_ref[...], wih_ref[...], ((1,), (1,))) + b_ref[...]
    bf = jnp.bfloat16
    wide_ref[:, 0:4 * _C + 2] = hw_ref[...].astype(bf)
    wide_ref[:, 4 * _C + 2:_H] = jnp.zeros((_H, _H - (4 * _C + 2)), bf)
    wide_ref[:, _H:2 * _H] = att_ref[...].astype(bf)
    wide_ref[:, 2 * _H:] = cc_ref[...].astype(bf)
    whh_o[...] = whh_ref[...].astype(bf)
    w1_o[...] = w1_ref[0:2 * _H, :].astype(bf)
    w2_o[...] = w2_ref[...].astype(bf)
    o1_o[...] = o1_ref[0:3 * _H, :].astype(bf)


_NT = 3200


@functools.partial(jax.jit, static_argnames=("interpret",))
def _run(x_flat, maskS, wihT, bih2, whhT, bhh2, w1lc, w1s, b1_2, ctrl_w2,
         b2_2, hw, hb, cc_w, cc_b2, att_w, attb2, o1m, o1s, ob1, out_w2,
         ob2, interpret=False):
    const = lambda s: (0, 0)
    bf = jnp.bfloat16
    gi_all, hw_wide, whh_bf, w1lc_bf, w2_bf, o1m_bf = pl.pallas_call(
        _gi_kernel,
        out_shape=[jax.ShapeDtypeStruct((_S * _B, 3 * _H), jnp.float32),
                   jax.ShapeDtypeStruct((_H, 2 * _H + _C * _H), jnp.bfloat16),
                   jax.ShapeDtypeStruct((3 * _H, _H), jnp.bfloat16),
                   jax.ShapeDtypeStruct((2 * _H, _H), jnp.bfloat16),
                   jax.ShapeDtypeStruct((_H, _H), jnp.bfloat16),
                   jax.ShapeDtypeStruct((3 * _H, _H), jnp.bfloat16)],
        name="cfrm_gi",
        interpret=interpret,
    )(x_flat, wihT, bih2, hw, att_w, cc_w, whhT, w1lc, ctrl_w2, o1m)

    h1_sb = pl.pallas_call(
        _scan_kernel,
        out_shape=jax.ShapeDtypeStruct((_B, _S * _H), jnp.float32),
        grid=(_S + 1,),
        in_specs=[
            pl.BlockSpec((_S * _B, 3 * _H), const),
            pl.BlockSpec((1, _B, 1),
                         lambda s: (jnp.minimum(s, _S - 1), 0, 0)),
            pl.BlockSpec((3 * _H, _H), const),
            pl.BlockSpec((1, 3 * _H), const),
            pl.BlockSpec((2 * _H, _H), const),
            pl.BlockSpec((4, _H), const),
            pl.BlockSpec((1, _H), const),
            pl.BlockSpec((_H, _H), const),
            pl.BlockSpec((1, _H), const),
            pl.BlockSpec((_H, 2 * _H + _C * _H), const),
            pl.BlockSpec((1, 4 * _C + 2), const),
            pl.BlockSpec((1, _C * _H), const),
            pl.BlockSpec((1, _H), const),
            pl.BlockSpec((3 * _H, _H), const),
            pl.BlockSpec((4, _H), const),
            pl.BlockSpec((1, _H), const),
        ],
        out_specs=pl.BlockSpec((_B, _H),
                               lambda s: (0, jnp.maximum(s - 1, 0))),
        scratch_shapes=[
            pltpu.VMEM((_BC, _H), jnp.float32),
            pltpu.VMEM((_B, _C), jnp.float32),
            pltpu.VMEM((_B, _C), jnp.float32),
            pltpu.VMEM((_B, _H), jnp.float32),
            pltpu.VMEM((_BC, _H), jnp.float32),
            pltpu.VMEM((_B, _H), jnp.float32),
            pltpu.VMEM((_B, 8), jnp.float32),
            pltpu.VMEM((_B, _C), jnp.float32),
            pltpu.VMEM((_B, _H), jnp.float32),
        ],
        compiler_params=pltpu.CompilerParams(
            dimension_semantics=("arbitrary",),
            vmem_limit_bytes=50 * 1024 * 1024,
        ),
        name="cfrm_scan",
        interpret=interpret,
    )(gi_all, maskS, whh_bf, bhh2, w1lc_bf, w1s, b1_2,
      w2_bf, b2_2, hw_wide, hb, cc_b2,
      attb2, o1m_bf, o1s, ob1)

    h1_bs = h1_sb.reshape(_B * _S, _H)
    logits = pl.pallas_call(
        _logits_kernel,
        out_shape=jax.ShapeDtypeStruct((_B * _S, _V), jnp.float32),
        grid=(_V // _NT,),
        in_specs=[
            pl.BlockSpec((_B * _S, _H), lambda j: (0, 0)),
            pl.BlockSpec((_H, _NT), lambda j: (0, j)),
            pl.BlockSpec((1, _NT), lambda j: (0, j)),
        ],
        out_specs=pl.BlockSpec((_B * _S, _NT), lambda j: (0, j)),
        compiler_params=pltpu.CompilerParams(
            dimension_semantics=("parallel",),
            vmem_limit_bytes=48 * 1024 * 1024,
        ),
        name="cfrm_logits",
        interpret=interpret,
    )(h1_bs.astype(bf), out_w2.astype(bf), ob2)
    return logits.reshape(_B, _S, _V)


def kernel(tokens, emb, gru_wih, gru_whh, gru_bih, gru_bhh, ctrl_w1, ctrl_b1,
           ctrl_w2, ctrl_b2, gate_w, gate_b, assign_w, assign_b, nov_w, nov_b,
           relax_w, relax_b, cc_w, cc_b, cs_w, cs_b, md_w, md_b, att_w, att_b,
           out_w1, out_b1, out_w2, out_b2, *, interpret=False):
    x = emb[tokens]
    x_flat = jnp.swapaxes(x, 0, 1).reshape(_S * _B, _E)
    maskS = (tokens != 0).astype(jnp.float32).T[:, :, None]
    hw = jnp.concatenate([gate_w, assign_w, cs_w, md_w, nov_w, relax_w], axis=1)
    hb = jnp.concatenate([gate_b, assign_b, cs_b, md_b, nov_b, relax_b])[None, :]
    return _run(x_flat, maskS, gru_wih, gru_bih[None, :], gru_whh,
                gru_bhh[None, :], ctrl_w1, ctrl_w1[2 * _H:],
                ctrl_b1[None, :], ctrl_w2, ctrl_b2[None, :], hw, hb,
                cc_w, cc_b[None, :], att_w, att_b[None, :],
                out_w1, out_w1[3 * _H:], out_b1[None, :],
                out_w2, out_b2[None, :], interpret=interpret)


# speedup vs baseline: 14.6795x; 1.1088x over previous
import functools
import math

import jax
import jax.numpy as jnp
from jax import lax
from jax.experimental import pallas as pl
from jax.experimental.pallas import tpu as pltpu

_V, _C, _H, _E = 32000, 32, 256, 256
_B, _S = 8, 128
_EPS = 1e-4
_BC = _B * _C
_NEG = -1e30
_INVSQRT2 = 1.0 / math.sqrt(2.0)


def _dot(a, b):
    return jnp.dot(a, b, preferred_element_type=jnp.float32)


def _dg(a, b, dims):
    return lax.dot_general(a, b, (dims, ((), ())),
                           preferred_element_type=jnp.float32)


def _split(a):
    hi = a.astype(jnp.bfloat16).astype(jnp.float32)
    return hi, a - hi


def _xl(a, b):
    hi, lo = _split(a)
    return _dot(hi, b) + _dot(lo, b)


def _xr(a, b):
    hi, lo = _split(b)
    return _dot(a, hi) + _dot(a, lo)


def _consts():
    f32 = jnp.float32
    i32 = jnp.int32
    bmask8 = (lax.broadcasted_iota(i32, (_B, _BC), 1) % _B
              == lax.broadcasted_iota(i32, (_B, _BC), 0)).astype(f32)
    Q = (lax.broadcasted_iota(i32, (_C, _BC), 1) // _B
         == lax.broadcasted_iota(i32, (_C, _BC), 0)).astype(f32)
    P = (lax.broadcasted_iota(i32, (_BC, _C), 0) // _B
         == lax.broadcasted_iota(i32, (_BC, _C), 1)).astype(f32)
    eexp = (lax.broadcasted_iota(i32, (_BC, _B), 0) % _B
            == lax.broadcasted_iota(i32, (_BC, _B), 1)).astype(f32)
    eye = (lax.broadcasted_iota(i32, (_BC, _BC), 0)
           == lax.broadcasted_iota(i32, (_BC, _BC), 1)).astype(f32)
    same = (lax.broadcasted_iota(i32, (_BC, _BC), 0) % _B
            == lax.broadcasted_iota(i32, (_BC, _BC), 1) % _B)
    return bmask8, Q, P, eexp, eye, same


def _to_row(m, Q, bmask8):
    return jnp.sum(_xl(m, Q) * bmask8, axis=0, keepdims=True)


def _row_to_col(r, eye):
    return _dg(eye, _split(r)[0], ((1,), (1,))) + \
        _dg(eye, r - _split(r)[0], ((1,), (1,)))


def _col_to_row(c, eye):
    hi, lo = _split(c)
    return _dg(hi, eye, ((0,), (0,))) + _dg(lo, eye, ((0,), (0,)))


def _row_to_m(r, P, bmask8):
    return _xl(jnp.broadcast_to(r, (_B, _BC)) * bmask8, P)


def _x2(a, b):
    ah, al = _split(a)
    bh, bl = _split(b)
    return _dot(ah, bh) + _dot(ah, bl) + _dot(al, bh)


def _to_col(m, eexp, P):
    return jnp.sum(_xr(eexp, m) * P, axis=-1, keepdims=True)


def _summarize(c2d, sp, ma, bmask8, Q, eexp, eye):
    prec = 1.0 / (sp + _EPS)
    scores = ma + jnp.log(prec + _EPS)
    smax = jnp.max(scores, axis=-1, keepdims=True)
    e = jnp.exp(scores - smax)
    alpha = e / jnp.sum(e, axis=-1, keepdims=True)
    A = _xl(alpha, Q) * bmask8
    core = _dot(A, c2d)
    unc = jnp.sum(alpha * sp, axis=-1, keepdims=True)
    sqn = jnp.sum(c2d * c2d, axis=-1, keepdims=True)
    div = (_x2(A, sqn)
           - jnp.sum(core * core, axis=-1, keepdims=True)) * (1.0 / _H)
    mmax = jnp.max(ma, axis=-1, keepdims=True)
    en = jnp.log(jnp.sum(jnp.exp(ma - mmax), axis=-1, keepdims=True)) + mmax
    ent = -jnp.sum(alpha * jnp.log(jnp.maximum(alpha, 1e-8)),
                   axis=-1, keepdims=True)
    return core, unc, div, en, ent, alpha


def _scan_kernel(gi_ref, mask_ref, whhT_ref, bhh_ref,
                 w1lc_ref, w1s_ref, b1_ref, w2_ref, b2_ref,
                 hw_ref, hb_ref, ccb_ref, attb_ref,
                 o1m_ref, o1s_ref, ob1_ref,
                 h1_ref,
                 c2d_ref, sp_ref, ma_ref, h_ref, cnd_ref, core_ref, scal_ref,
                 alpha_ref, lp_ref):
    s = pl.program_id(0)
    bmask8, Q, P, eexp, eye, same = _consts()

    @pl.when(s == 0)
    def _init():
        c2d_ref[...] = jnp.zeros_like(c2d_ref)
        sp_ref[...] = jnp.ones_like(sp_ref)
        ma_ref[...] = jnp.zeros_like(ma_ref)
        core_ref[...] = jnp.zeros_like(core_ref)
        lc = math.log(_C)
        col8 = lax.broadcasted_iota(jnp.int32, (_B, 8), 1)
        scal_ref[...] = jnp.where(
            col8 == 0, 1.0, jnp.where((col8 == 2) | (col8 == 3), lc, 0.0))
        alpha_ref[...] = jnp.full((_B, _C), 1.0 / _C, jnp.float32)
        lp_ref[...] = jnp.zeros_like(lp_ref)
        gi0 = gi_ref[0:_B, :]
        gh0 = jnp.broadcast_to(bhh_ref[...], (_B, 3 * _H))
        r0 = jax.nn.sigmoid(gi0[:, :_H] + gh0[:, :_H])
        z0 = jax.nn.sigmoid(gi0[:, _H:2 * _H] + gh0[:, _H:2 * _H])
        n0 = jnp.tanh(gi0[:, 2 * _H:] + r0 * gh0[:, 2 * _H:])
        h_ref[...] = (1.0 - z0) * n0

    valid = mask_ref[0]
    local = h_ref[...]
    c2d = c2d_ref[...]
    sp = sp_ref[...]
    ma = ma_ref[...]
    core = core_ref[...]
    unc = scal_ref[:, 0:1]
    div = scal_ref[:, 1:2]
    en = scal_ref[:, 2:3]
    ent = scal_ref[:, 3:4]

    c2d0 = c2d

    pre = _dot(jnp.concatenate([local, core], axis=1).astype(jnp.bfloat16),
               w1lc_ref[...]) + b1_ref[...]
    pre = (pre + unc * w1s_ref[0:1, :] + div * w1s_ref[1:2, :]
           + en * w1s_ref[2:3, :] + ent * w1s_ref[3:4, :])
    ctrl = jnp.tanh(_dot(jnp.tanh(pre).astype(jnp.bfloat16), w2_ref[...])
                    + b2_ref[...])
    ctrl_bf = ctrl.astype(jnp.bfloat16)

    y = _dot(ctrl_bf, hw_ref[...])
    hs = y[:, 0:4 * _C + 2] + hb_ref[...]
    attr = y[:, _H:2 * _H] + attb_ref[...]
    cand = y[:, 2 * _H:] + ccb_ref[...]
    gate = jax.nn.sigmoid(hs[:, 0:_C]) * valid
    ae = hs[:, _C:2 * _C]
    ae = jnp.exp(ae - jnp.max(ae, axis=-1, keepdims=True))
    assign = ae / jnp.sum(ae, axis=-1, keepdims=True)
    cs_raw = hs[:, 2 * _C:3 * _C]
    cand_sp = (jnp.maximum(cs_raw, 0.0)
               + jnp.log(1.0 + jnp.exp(-jnp.abs(cs_raw))) + _EPS)
    mdel = jnp.tanh(hs[:, 3 * _C:4 * _C])
    nov = jax.nn.sigmoid(hs[:, 4 * _C:4 * _C + 1]) * valid
    relax = jax.nn.sigmoid(hs[:, 4 * _C + 1:4 * _C + 2]) * valid

    for c in range(_C):
        cnd_ref[c * _B:(c + 1) * _B, :] = cand[:, c * _H:(c + 1) * _H]
    cand2d = cnd_ref[...]

    strength = gate * assign
    st_col = _to_col(strength, eexp, P)
    c2d = c2d + st_col * (cand2d - c2d)
    sp = sp + strength * (cand_sp - sp)
    ma = ma + strength * mdel

    anr = _xr(eexp, jnp.concatenate([attr, nov, relax], axis=1))
    attr_exp = anr[:, :_H]
    nov_col = anr[:, _H:_H + 1]
    relax_col = anr[:, _H + 1:_H + 2]
    c2d = c2d + 0.1 * nov_col * (attr_exp - c2d)

    sp_row = _to_row(sp, Q, bmask8)
    ma_row = _to_row(ma, Q, bmask8)
    sp_col = _to_col(sp, eexp, P)
    sq_col = jnp.sum(c2d * c2d, axis=-1, keepdims=True)
    G = _dg(c2d, c2d, ((1,), (1,)))
    sq_row = jnp.sum(G * eye, axis=0, keepdims=True)
    d2 = jnp.maximum(sq_col + sq_row - 2.0 * G, 0.0)
    scale = sp_col + sp_row + _EPS
    compat = jnp.where(same, -d2 / scale + ma_row, _NEG)
    alpha2 = alpha_ref[...]
    cidx = lax.broadcasted_iota(jnp.int32, (_B, _C), 1)
    amax = jnp.max(alpha2, axis=-1, keepdims=True)
    idx = jnp.min(jnp.where(alpha2 == amax, cidx, _C), axis=-1, keepdims=True)
    jj = lax.broadcasted_iota(jnp.int32, (_B, _BC), 1)
    bb = lax.broadcasted_iota(jnp.int32, (_B, _BC), 0)
    oh = (((jj // _B) == idx) & ((jj % _B) == bb)).astype(jnp.float32)
    strongest = _dot(oh, c2d0)
    feat = jnp.concatenate([lp_ref[...], core, strongest], axis=1)
    h1 = _dot(feat.astype(jnp.bfloat16), o1m_ref[...]) + ob1_ref[...]
    h1 = (h1 + unc * o1s_ref[0:1, :] + div * o1s_ref[1:2, :]
          + en * o1s_ref[2:3, :] + ent * o1s_ref[3:4, :])
    h1 = 0.5 * h1 * (1.0 + lax.erf(h1 * _INVSQRT2))
    h1_ref[...] = h1

    cmax = jnp.max(compat, axis=-1, keepdims=True)
    cexp = jnp.exp(compat - cmax)
    mixing = cexp / jnp.sum(cexp, axis=-1, keepdims=True)
    mc = _dot(mixing, c2d)
    msp = jnp.sum(mixing * sp_row, axis=-1, keepdims=True)
    mma = jnp.sum(mixing * ma_row, axis=-1, keepdims=True)

    c2d = (1.0 - relax_col) * c2d + relax_col * mc
    mrows = _col_to_row(jnp.concatenate([msp, mma], axis=1), eye)
    sp = (1.0 - relax) * sp + relax * _row_to_m(mrows[0:1], P, bmask8)
    ma = (1.0 - relax) * ma + relax * _row_to_m(mrows[1:2], P, bmask8)

    c2d_ref[...] = c2d
    sp_ref[...] = sp
    ma_ref[...] = ma
    lp_ref[...] = local

    sn = jnp.minimum(s + 1, _S - 1)
    gi = gi_ref[pl.ds(pl.multiple_of(sn * _B, _B), _B), :]
    gh = _dg(local.astype(jnp.bfloat16), whhT_ref[...], ((1,), (1,))) + bhh_ref[...]
    r = jax.nn.sigmoid(gi[:, :_H] + gh[:, :_H])
    z = jax.nn.sigmoid(gi[:, _H:2 * _H] + gh[:, _H:2 * _H])
    n = jnp.tanh(gi[:, 2 * _H:] + r * gh[:, 2 * _H:])
    h_ref[...] = (1.0 - z) * n + z * local


    core2, unc2, div2, en2, ent2, alpha2 = _summarize(
        c2d, sp, ma, bmask8, Q, eexp, eye)
    core_ref[...] = core2
    scal_ref[...] = jnp.concatenate(
        [unc2, div2, en2, ent2, jnp.zeros((_B, 4), jnp.float32)], axis=1)
    alpha_ref[...] = alpha2


def _logits_kernel(a_ref, w_ref, b_ref, o_ref):
    o_ref[...] = _dot(a_ref[...], w_ref[...]) + b_ref[...]


def _gi_kernel(x_ref, wih_ref, b_ref, hw_ref, att_ref, cc_ref, whh_ref,
               w1_ref, w2_ref, o1_ref,
               gi_ref, wide_ref, whh_o, w1_o, w2_o, o1_o):
    gi_ref[...] = _dg(x_ref[...], wih_ref[...], ((1,), (1,))) + b_ref[...]
    bf = jnp.bfloat16
    wide_ref[:, 0:4 * _C + 2] = hw_ref[...].astype(bf)
    wide_ref[:, 4 * _C + 2:_H] = jnp.zeros((_H, _H - (4 * _C + 2)), bf)
    wide_ref[:, _H:2 * _H] = att_ref[...].astype(bf)
    wide_ref[:, 2 * _H:] = cc_ref[...].astype(bf)
    whh_o[...] = whh_ref[...].astype(bf)
    w1_o[...] = w1_ref[0:2 * _H, :].astype(bf)
    w2_o[...] = w2_ref[...].astype(bf)
    o1_o[...] = o1_ref[0:3 * _H, :].astype(bf)


_NT = 3200


@functools.partial(jax.jit, static_argnames=("interpret",))
def _run(x_flat, maskS, wihT, bih2, whhT, bhh2, w1lc, w1s, b1_2, ctrl_w2,
         b2_2, hw, hb, cc_w, cc_b2, att_w, attb2, o1m, o1s, ob1, out_w2,
         ob2, interpret=False):
    const = lambda s: (0, 0)
    bf = jnp.bfloat16
    gi_all, hw_wide, whh_bf, w1lc_bf, w2_bf, o1m_bf = pl.pallas_call(
        _gi_kernel,
        out_shape=[jax.ShapeDtypeStruct((_S * _B, 3 * _H), jnp.float32),
                   jax.ShapeDtypeStruct((_H, 2 * _H + _C * _H), jnp.bfloat16),
                   jax.ShapeDtypeStruct((3 * _H, _H), jnp.bfloat16),
                   jax.ShapeDtypeStruct((2 * _H, _H), jnp.bfloat16),
                   jax.ShapeDtypeStruct((_H, _H), jnp.bfloat16),
                   jax.ShapeDtypeStruct((3 * _H, _H), jnp.bfloat16)],
        name="cfrm_gi",
        interpret=interpret,
    )(x_flat, wihT, bih2, hw, att_w, cc_w, whhT, w1lc, ctrl_w2, o1m)

    h1_sb = pl.pallas_call(
        _scan_kernel,
        out_shape=jax.ShapeDtypeStruct((_B, _S * _H), jnp.float32),
        grid=(_S + 1,),
        in_specs=[
            pl.BlockSpec((_S * _B, 3 * _H), const),
            pl.BlockSpec((1, _B, 1),
                         lambda s: (jnp.minimum(s, _S - 1), 0, 0)),
            pl.BlockSpec((3 * _H, _H), const),
            pl.BlockSpec((1, 3 * _H), const),
            pl.BlockSpec((2 * _H, _H), const),
            pl.BlockSpec((4, _H), const),
            pl.BlockSpec((1, _H), const),
            pl.BlockSpec((_H, _H), const),
            pl.BlockSpec((1, _H), const),
            pl.BlockSpec((_H, 2 * _H + _C * _H), const),
            pl.BlockSpec((1, 4 * _C + 2), const),
            pl.BlockSpec((1, _C * _H), const),
            pl.BlockSpec((1, _H), const),
            pl.BlockSpec((3 * _H, _H), const),
            pl.BlockSpec((4, _H), const),
            pl.BlockSpec((1, _H), const),
        ],
        out_specs=pl.BlockSpec((_B, _H),
                               lambda s: (0, jnp.maximum(s - 1, 0))),
        scratch_shapes=[
            pltpu.VMEM((_BC, _H), jnp.float32),
            pltpu.VMEM((_B, _C), jnp.float32),
            pltpu.VMEM((_B, _C), jnp.float32),
            pltpu.VMEM((_B, _H), jnp.float32),
            pltpu.VMEM((_BC, _H), jnp.float32),
            pltpu.VMEM((_B, _H), jnp.float32),
            pltpu.VMEM((_B, 8), jnp.float32),
            pltpu.VMEM((_B, _C), jnp.float32),
            pltpu.VMEM((_B, _H), jnp.float32),
        ],
        compiler_params=pltpu.CompilerParams(
            dimension_semantics=("arbitrary",),
            vmem_limit_bytes=50 * 1024 * 1024,
        ),
        name="cfrm_scan",
        interpret=interpret,
    )(gi_all, maskS, whh_bf, bhh2, w1lc_bf, w1s, b1_2,
      w2_bf, b2_2, hw_wide, hb, cc_b2,
      attb2, o1m_bf, o1s, ob1)

    h1_bs = h1_sb.reshape(_B * _S, _H)
    logits = pl.pallas_call(
        _logits_kernel,
        out_shape=jax.ShapeDtypeStruct((_B * _S, _V), jnp.float32),
        grid=(_V // _NT,),
        in_specs=[
            pl.BlockSpec((_B * _S, _H), lambda j: (0, 0)),
            pl.BlockSpec((_H, _NT), lambda j: (0, j)),
            pl.BlockSpec((1, _NT), lambda j: (0, j)),
        ],
        out_specs=pl.BlockSpec((_B * _S, _NT), lambda j: (0, j)),
        compiler_params=pltpu.CompilerParams(
            dimension_semantics=("parallel",),
            vmem_limit_bytes=48 * 1024 * 1024,
        ),
        name="cfrm_logits",
        interpret=interpret,
    )(h1_bs.astype(bf), out_w2.astype(bf), ob2)
    return logits.reshape(_B, _S, _V)


def kernel(tokens, emb, gru_wih, gru_whh, gru_bih, gru_bhh, ctrl_w1, ctrl_b1,
           ctrl_w2, ctrl_b2, gate_w, gate_b, assign_w, assign_b, nov_w, nov_b,
           relax_w, relax_b, cc_w, cc_b, cs_w, cs_b, md_w, md_b, att_w, att_b,
           out_w1, out_b1, out_w2, out_b2, *, interpret=False):
    x = emb[tokens]
    x_flat = jnp.swapaxes(x, 0, 1).reshape(_S * _B, _E)
    maskS = (tokens != 0).astype(jnp.float32).T[:, :, None]
    hw = jnp.concatenate([gate_w, assign_w, cs_w, md_w, nov_w, relax_w], axis=1)
    hb = jnp.concatenate([gate_b, assign_b, cs_b, md_b, nov_b, relax_b])[None, :]
    return _run(x_flat, maskS, gru_wih, gru_bih[None, :], gru_whh,
                gru_bhh[None, :], ctrl_w1, ctrl_w1[2 * _H:],
                ctrl_b1[None, :], ctrl_w2, ctrl_b2[None, :], hw, hb,
                cc_w, cc_b[None, :], att_w, att_b[None, :],
                out_w1, out_w1[3 * _H:], out_b1[None, :],
                out_w2, out_b2[None, :], interpret=interpret)


# speedup vs baseline: 14.6795x; 1.0000x over previous
import functools
import math

import jax
import jax.numpy as jnp
from jax import lax
from jax.experimental import pallas as pl
from jax.experimental.pallas import tpu as pltpu

_V, _C, _H, _E = 32000, 32, 256, 256
_B, _S = 8, 128
_EPS = 1e-4
_BC = _B * _C
_NEG = -1e30
_INVSQRT2 = 1.0 / math.sqrt(2.0)


def _dot(a, b):
    return jnp.dot(a, b, preferred_element_type=jnp.float32)


def _dg(a, b, dims):
    return lax.dot_general(a, b, (dims, ((), ())),
                           preferred_element_type=jnp.float32)


def _split(a):
    hi = a.astype(jnp.bfloat16).astype(jnp.float32)
    return hi, a - hi


def _xl(a, b):
    hi, lo = _split(a)
    return _dot(hi, b) + _dot(lo, b)


def _xr(a, b):
    hi, lo = _split(b)
    return _dot(a, hi) + _dot(a, lo)


def _consts():
    f32 = jnp.float32
    i32 = jnp.int32
    bmask8 = (lax.broadcasted_iota(i32, (_B, _BC), 1) % _B
              == lax.broadcasted_iota(i32, (_B, _BC), 0)).astype(f32)
    Q = (lax.broadcasted_iota(i32, (_C, _BC), 1) // _B
         == lax.broadcasted_iota(i32, (_C, _BC), 0)).astype(f32)
    P = (lax.broadcasted_iota(i32, (_BC, _C), 0) // _B
         == lax.broadcasted_iota(i32, (_BC, _C), 1)).astype(f32)
    eexp = (lax.broadcasted_iota(i32, (_BC, _B), 0) % _B
            == lax.broadcasted_iota(i32, (_BC, _B), 1)).astype(f32)
    eye = (lax.broadcasted_iota(i32, (_BC, _BC), 0)
           == lax.broadcasted_iota(i32, (_BC, _BC), 1)).astype(f32)
    same = (lax.broadcasted_iota(i32, (_BC, _BC), 0) % _B
            == lax.broadcasted_iota(i32, (_BC, _BC), 1) % _B)
    return bmask8, Q, P, eexp, eye, same


def _to_row(m, Q, bmask8):
    return jnp.sum(_xl(m, Q) * bmask8, axis=0, keepdims=True)


def _row_to_col(r, eye):
    return _dg(eye, _split(r)[0], ((1,), (1,))) + \
        _dg(eye, r - _split(r)[0], ((1,), (1,)))


def _col_to_row(c, eye):
    hi, lo = _split(c)
    return _dg(hi, eye, ((0,), (0,))) + _dg(lo, eye, ((0,), (0,)))


def _row_to_m(r, P, bmask8):
    return _xl(jnp.broadcast_to(r, (_B, _BC)) * bmask8, P)


def _x2(a, b):
    ah, al = _split(a)
    bh, bl = _split(b)
    return _dot(ah, bh) + _dot(ah, bl) + _dot(al, bh)


def _to_col(m, eexp, P):
    return jnp.sum(_xr(eexp, m) * P, axis=-1, keepdims=True)


def _summarize(c2d, sp, ma, bmask8, Q, eexp, eye):
    prec = 1.0 / (sp + _EPS)
    scores = ma + jnp.log(prec + _EPS)
    smax = jnp.max(scores, axis=-1, keepdims=True)
    e = jnp.exp(scores - smax)
    alpha = e / jnp.sum(e, axis=-1, keepdims=True)
    A = _xl(alpha, Q) * bmask8
    core = _dot(A, c2d)
    unc = jnp.sum(alpha * sp, axis=-1, keepdims=True)
    sqn = jnp.sum(c2d * c2d, axis=-1, keepdims=True)
    div = (_x2(A, sqn)
           - jnp.sum(core * core, axis=-1, keepdims=True)) * (1.0 / _H)
    mmax = jnp.max(ma, axis=-1, keepdims=True)
    en = jnp.log(jnp.sum(jnp.exp(ma - mmax), axis=-1, keepdims=True)) + mmax
    ent = -jnp.sum(alpha * jnp.log(jnp.maximum(alpha, 1e-8)),
                   axis=-1, keepdims=True)
    return core, unc, div, en, ent, alpha


def _scan_kernel(gi_ref, mask_ref, whhT_ref, bhh_ref,
                 w1lc_ref, w1s_ref, b1_ref, w2_ref, b2_ref,
                 hw_ref, hb_ref, ccb_ref, attb_ref,
                 o1m_ref, o1s_ref, ob1_ref,
                 h1_ref,
                 c2d_ref, sp_ref, ma_ref, h_ref, cnd_ref, core_ref, scal_ref,
                 alpha_ref, lp_ref):
    s = pl.program_id(0)
    bmask8, Q, P, eexp, eye, same = _consts()

    @pl.when(s == 0)
    def _init():
        c2d_ref[...] = jnp.zeros_like(c2d_ref)
        sp_ref[...] = jnp.ones_like(sp_ref)
        ma_ref[...] = jnp.zeros_like(ma_ref)
        core_ref[...] = jnp.zeros_like(core_ref)
        lc = math.log(_C)
        col8 = lax.broadcasted_iota(jnp.int32, (_B, 8), 1)
        scal_ref[...] = jnp.where(
            col8 == 0, 1.0, jnp.where((col8 == 2) | (col8 == 3), lc, 0.0))
        alpha_ref[...] = jnp.full((_B, _C), 1.0 / _C, jnp.float32)
        lp_ref[...] = jnp.zeros_like(lp_ref)
        gi0 = gi_ref[0:_B, :]
        gh0 = jnp.broadcast_to(bhh_ref[...], (_B, 3 * _H))
        r0 = jax.nn.sigmoid(gi0[:, :_H] + gh0[:, :_H])
        z0 = jax.nn.sigmoid(gi0[:, _H:2 * _H] + gh0[:, _H:2 * _H])
        n0 = jnp.tanh(gi0[:, 2 * _H:] + r0 * gh0[:, 2 * _H:])
        h_ref[...] = (1.0 - z0) * n0

    valid = mask_ref[0]
    local = h_ref[...]
    c2d = c2d_ref[...]
    sp = sp_ref[...]
    ma = ma_ref[...]
    core = core_ref[...]
    unc = scal_ref[:, 0:1]
    div = scal_ref[:, 1:2]
    en = scal_ref[:, 2:3]
    ent = scal_ref[:, 3:4]

    c2d0 = c2d

    pre = _dot(jnp.concatenate([local, core], axis=1).astype(jnp.bfloat16),
               w1lc_ref[...]) + b1_ref[...]
    pre = (pre + unc * w1s_ref[0:1, :] + div * w1s_ref[1:2, :]
           + en * w1s_ref[2:3, :] + ent * w1s_ref[3:4, :])
    ctrl = jnp.tanh(_dot(jnp.tanh(pre).astype(jnp.bfloat16), w2_ref[...])
                    + b2_ref[...])
    ctrl_bf = ctrl.astype(jnp.bfloat16)

    y = _dot(ctrl_bf, hw_ref[...])
    hs = y[:, 0:4 * _C + 2] + hb_ref[...]
    attr = y[:, _H:2 * _H] + attb_ref[...]
    cand = y[:, 2 * _H:] + ccb_ref[...]
    gate = jax.nn.sigmoid(hs[:, 0:_C]) * valid
    ae = hs[:, _C:2 * _C]
    ae = jnp.exp(ae - jnp.max(ae, axis=-1, keepdims=True))
    assign = ae / jnp.sum(ae, axis=-1, keepdims=True)
    cs_raw = hs[:, 2 * _C:3 * _C]
    cand_sp = (jnp.maximum(cs_raw, 0.0)
               + jnp.log(1.0 + jnp.exp(-jnp.abs(cs_raw))) + _EPS)
    mdel = jnp.tanh(hs[:, 3 * _C:4 * _C])
    nov = jax.nn.sigmoid(hs[:, 4 * _C:4 * _C + 1]) * valid
    relax = jax.nn.sigmoid(hs[:, 4 * _C + 1:4 * _C + 2]) * valid

    for c in range(_C):
        cnd_ref[c * _B:(c + 1) * _B, :] = cand[:, c * _H:(c + 1) * _H]
    cand2d = cnd_ref[...]

    strength = gate * assign
    st_col = _to_col(strength, eexp, P)
    c2d = c2d + st_col * (cand2d - c2d)
    sp = sp + strength * (cand_sp - sp)
    ma = ma + strength * mdel

    anr = _xr(eexp, jnp.concatenate([attr, nov, relax], axis=1))
    attr_exp = anr[:, :_H]
    nov_col = anr[:, _H:_H + 1]
    relax_col = anr[:, _H + 1:_H + 2]
    c2d = c2d + 0.1 * nov_col * (attr_exp - c2d)

    sp_row = _to_row(sp, Q, bmask8)
    ma_row = _to_row(ma, Q, bmask8)
    sp_col = _to_col(sp, eexp, P)
    sq_col = jnp.sum(c2d * c2d, axis=-1, keepdims=True)
    c2d_bf = c2d.astype(jnp.bfloat16)
    G = _dg(c2d_bf, c2d_bf, ((1,), (1,)))
    sq_row = jnp.sum(G * eye, axis=0, keepdims=True)
    d2 = jnp.maximum(sq_col + sq_row - 2.0 * G, 0.0)
    scale = sp_col + sp_row + _EPS
    compat = jnp.where(same, -d2 / scale + ma_row, _NEG)
    alpha2 = alpha_ref[...]
    cidx = lax.broadcasted_iota(jnp.int32, (_B, _C), 1)
    amax = jnp.max(alpha2, axis=-1, keepdims=True)
    idx = jnp.min(jnp.where(alpha2 == amax, cidx, _C), axis=-1, keepdims=True)
    jj = lax.broadcasted_iota(jnp.int32, (_B, _BC), 1)
    bb = lax.broadcasted_iota(jnp.int32, (_B, _BC), 0)
    oh = (((jj // _B) == idx) & ((jj % _B) == bb)).astype(jnp.float32)
    strongest = _dot(oh, c2d0)
    feat = jnp.concatenate([lp_ref[...], core, strongest], axis=1)
    h1 = _dot(feat.astype(jnp.bfloat16), o1m_ref[...]) + ob1_ref[...]
    h1 = (h1 + unc * o1s_ref[0:1, :] + div * o1s_ref[1:2, :]
          + en * o1s_ref[2:3, :] + ent * o1s_ref[3:4, :])
    h1 = 0.5 * h1 * (1.0 + lax.erf(h1 * _INVSQRT2))
    h1_ref[...] = h1

    cmax = jnp.max(compat, axis=-1, keepdims=True)
    cexp = jnp.exp(compat - cmax)
    mixing = cexp / jnp.sum(cexp, axis=-1, keepdims=True)
    mc = _dot(mixing.astype(jnp.bfloat16), c2d_bf)
    msp = jnp.sum(mixing * sp_row, axis=-1, keepdims=True)
    mma = jnp.sum(mixing * ma_row, axis=-1, keepdims=True)

    c2d = (1.0 - relax_col) * c2d + relax_col * mc
    mrows = _col_to_row(jnp.concatenate([msp, mma], axis=1), eye)
    sp = (1.0 - relax) * sp + relax * _row_to_m(mrows[0:1], P, bmask8)
    ma = (1.0 - relax) * ma + relax * _row_to_m(mrows[1:2], P, bmask8)

    c2d_ref[...] = c2d
    sp_ref[...] = sp
    ma_ref[...] = ma
    lp_ref[...] = local

    sn = jnp.minimum(s + 1, _S - 1)
    gi = gi_ref[pl.ds(pl.multiple_of(sn * _B, _B), _B), :]
    gh = _dg(local.astype(jnp.bfloat16), whhT_ref[...], ((1,), (1,))) + bhh_ref[...]
    r = jax.nn.sigmoid(gi[:, :_H] + gh[:, :_H])
    z = jax.nn.sigmoid(gi[:, _H:2 * _H] + gh[:, _H:2 * _H])
    n = jnp.tanh(gi[:, 2 * _H:] + r * gh[:, 2 * _H:])
    h_ref[...] = (1.0 - z) * n + z * local


    core2, unc2, div2, en2, ent2, alpha2 = _summarize(
        c2d, sp, ma, bmask8, Q, eexp, eye)
    core_ref[...] = core2
    scal_ref[...] = jnp.concatenate(
        [unc2, div2, en2, ent2, jnp.zeros((_B, 4), jnp.float32)], axis=1)
    alpha_ref[...] = alpha2


def _logits_kernel(a_ref, w_ref, b_ref, o_ref):
    o_ref[...] = _dot(a_ref[...], w_ref[...]) + b_ref[...]


def _gi_kernel(x_ref, wih_ref, b_ref, hw_ref, att_ref, cc_ref, whh_ref,
               w1_ref, w2_ref, o1_ref,
               gi_ref, wide_ref, whh_o, w1_o, w2_o, o1_o):
    gi_ref[...] = _dg(x_ref[...], wih_ref[...], ((1,), (1,))) + b_ref[...]
    bf = jnp.bfloat16
    wide_ref[:, 0:4 * _C + 2] = hw_ref[...].astype(bf)
    wide_ref[:, 4 * _C + 2:_H] = jnp.zeros((_H, _H - (4 * _C + 2)), bf)
    wide_ref[:, _H:2 * _H] = att_ref[...].astype(bf)
    wide_ref[:, 2 * _H:] = cc_ref[...].astype(bf)
    whh_o[...] = whh_ref[...].astype(bf)
    w1_o[...] = w1_ref[0:2 * _H, :].astype(bf)
    w2_o[...] = w2_ref[...].astype(bf)
    o1_o[...] = o1_ref[0:3 * _H, :].astype(bf)


_NT = 3200


@functools.partial(jax.jit, static_argnames=("interpret",))
def _run(x_flat, maskS, wihT, bih2, whhT, bhh2, w1lc, w1s, b1_2, ctrl_w2,
         b2_2, hw, hb, cc_w, cc_b2, att_w, attb2, o1m, o1s, ob1, out_w2,
         ob2, interpret=False):
    const = lambda s: (0, 0)
    bf = jnp.bfloat16
    gi_all, hw_wide, whh_bf, w1lc_bf, w2_bf, o1m_bf = pl.pallas_call(
        _gi_kernel,
        out_shape=[jax.ShapeDtypeStruct((_S * _B, 3 * _H), jnp.float32),
                   jax.ShapeDtypeStruct((_H, 2 * _H + _C * _H), jnp.bfloat16),
                   jax.ShapeDtypeStruct((3 * _H, _H), jnp.bfloat16),
                   jax.ShapeDtypeStruct((2 * _H, _H), jnp.bfloat16),
                   jax.ShapeDtypeStruct((_H, _H), jnp.bfloat16),
                   jax.ShapeDtypeStruct((3 * _H, _H), jnp.bfloat16)],
        name="cfrm_gi",
        interpret=interpret,
    )(x_flat, wihT, bih2, hw, att_w, cc_w, whhT, w1lc, ctrl_w2, o1m)

    h1_sb = pl.pallas_call(
        _scan_kernel,
        out_shape=jax.ShapeDtypeStruct((_B, _S * _H), jnp.float32),
        grid=(_S + 1,),
        in_specs=[
            pl.BlockSpec((_S * _B, 3 * _H), const),
            pl.BlockSpec((1, _B, 1),
                         lambda s: (jnp.minimum(s, _S - 1), 0, 0)),
            pl.BlockSpec((3 * _H, _H), const),
            pl.BlockSpec((1, 3 * _H), const),
            pl.BlockSpec((2 * _H, _H), const),
            pl.BlockSpec((4, _H), const),
            pl.BlockSpec((1, _H), const),
            pl.BlockSpec((_H, _H), const),
            pl.BlockSpec((1, _H), const),
            pl.BlockSpec((_H, 2 * _H + _C * _H), const),
            pl.BlockSpec((1, 4 * _C + 2), const),
            pl.BlockSpec((1, _C * _H), const),
            pl.BlockSpec((1, _H), const),
            pl.BlockSpec((3 * _H, _H), const),
            pl.BlockSpec((4, _H), const),
            pl.BlockSpec((1, _H), const),
        ],
        out_specs=pl.BlockSpec((_B, _H),
                               lambda s: (0, jnp.maximum(s - 1, 0))),
        scratch_shapes=[
            pltpu.VMEM((_BC, _H), jnp.float32),
            pltpu.VMEM((_B, _C), jnp.float32),
            pltpu.VMEM((_B, _C), jnp.float32),
            pltpu.VMEM((_B, _H), jnp.float32),
            pltpu.VMEM((_BC, _H), jnp.float32),
            pltpu.VMEM((_B, _H), jnp.float32),
            pltpu.VMEM((_B, 8), jnp.float32),
            pltpu.VMEM((_B, _C), jnp.float32),
            pltpu.VMEM((_B, _H), jnp.float32),
        ],
        compiler_params=pltpu.CompilerParams(
            dimension_semantics=("arbitrary",),
            vmem_limit_bytes=50 * 1024 * 1024,
        ),
        name="cfrm_scan",
        interpret=interpret,
    )(gi_all, maskS, whh_bf, bhh2, w1lc_bf, w1s, b1_2,
      w2_bf, b2_2, hw_wide, hb, cc_b2,
      attb2, o1m_bf, o1s, ob1)

    h1_bs = h1_sb.reshape(_B * _S, _H)
    logits = pl.pallas_call(
        _logits_kernel,
        out_shape=jax.ShapeDtypeStruct((_B * _S, _V), jnp.float32),
        grid=(_V // _NT,),
        in_specs=[
            pl.BlockSpec((_B * _S, _H), lambda j: (0, 0)),
            pl.BlockSpec((_H, _NT), lambda j: (0, j)),
            pl.BlockSpec((1, _NT), lambda j: (0, j)),
        ],
        out_specs=pl.BlockSpec((_B * _S, _NT), lambda j: (0, j)),
        compiler_params=pltpu.CompilerParams(
            dimension_semantics=("parallel",),
            vmem_limit_bytes=48 * 1024 * 1024,
        ),
        name="cfrm_logits",
        interpret=interpret,
    )(h1_bs.astype(bf), out_w2.astype(bf), ob2)
    return logits.reshape(_B, _S, _V)


def kernel(tokens, emb, gru_wih, gru_whh, gru_bih, gru_bhh, ctrl_w1, ctrl_b1,
           ctrl_w2, ctrl_b2, gate_w, gate_b, assign_w, assign_b, nov_w, nov_b,
           relax_w, relax_b, cc_w, cc_b, cs_w, cs_b, md_w, md_b, att_w, att_b,
           out_w1, out_b1, out_w2, out_b2, *, interpret=False):
    x = emb[tokens]
    x_flat = jnp.swapaxes(x, 0, 1).reshape(_S * _B, _E)
    maskS = (tokens != 0).astype(jnp.float32).T[:, :, None]
    hw = jnp.concatenate([gate_w, assign_w, cs_w, md_w, nov_w, relax_w], axis=1)
    hb = jnp.concatenate([gate_b, assign_b, cs_b, md_b, nov_b, relax_b])[None, :]
    return _run(x_flat, maskS, gru_wih, gru_bih[None, :], gru_whh,
                gru_bhh[None, :], ctrl_w1, ctrl_w1[2 * _H:],
                ctrl_b1[None, :], ctrl_w2, ctrl_b2[None, :], hw, hb,
                cc_w, cc_b[None, :], att_w, att_b[None, :],
                out_w1, out_w1[3 * _H:], out_b1[None, :],
                out_w2, out_b2[None, :], interpret=interpret)


# speedup vs baseline: 14.6838x; 1.0003x over previous
import functools
import math

import jax
import jax.numpy as jnp
from jax import lax
from jax.experimental import pallas as pl
from jax.experimental.pallas import tpu as pltpu

_V, _C, _H, _E = 32000, 32, 256, 256
_B, _S = 8, 128
_EPS = 1e-4
_BC = _B * _C
_NEG = -1e30
_INVSQRT2 = 1.0 / math.sqrt(2.0)


def _dot(a, b):
    return jnp.dot(a, b, preferred_element_type=jnp.float32)


def _dg(a, b, dims):
    return lax.dot_general(a, b, (dims, ((), ())),
                           preferred_element_type=jnp.float32)


def _split(a):
    hi = a.astype(jnp.bfloat16).astype(jnp.float32)
    return hi, a - hi


def _xl(a, b):
    hi, lo = _split(a)
    return _dot(hi, b) + _dot(lo, b)


def _xr(a, b):
    hi, lo = _split(b)
    return _dot(a, hi) + _dot(a, lo)


def _consts():
    f32 = jnp.float32
    i32 = jnp.int32
    bmask8 = (lax.broadcasted_iota(i32, (_B, _BC), 1) % _B
              == lax.broadcasted_iota(i32, (_B, _BC), 0)).astype(f32)
    Q = (lax.broadcasted_iota(i32, (_C, _BC), 1) // _B
         == lax.broadcasted_iota(i32, (_C, _BC), 0)).astype(f32)
    P = (lax.broadcasted_iota(i32, (_BC, _C), 0) // _B
         == lax.broadcasted_iota(i32, (_BC, _C), 1)).astype(f32)
    eexp = (lax.broadcasted_iota(i32, (_BC, _B), 0) % _B
            == lax.broadcasted_iota(i32, (_BC, _B), 1)).astype(f32)
    eye = (lax.broadcasted_iota(i32, (_BC, _BC), 0)
           == lax.broadcasted_iota(i32, (_BC, _BC), 1)).astype(f32)
    same = (lax.broadcasted_iota(i32, (_BC, _BC), 0) % _B
            == lax.broadcasted_iota(i32, (_BC, _BC), 1) % _B)
    return bmask8, Q, P, eexp, eye, same


def _to_row(m, Q, bmask8):
    return jnp.sum(_xl(m, Q) * bmask8, axis=0, keepdims=True)


def _row_to_col(r, eye):
    return _dg(eye, _split(r)[0], ((1,), (1,))) + \
        _dg(eye, r - _split(r)[0], ((1,), (1,)))


def _col_to_row(c, eye):
    hi, lo = _split(c)
    return _dg(hi, eye, ((0,), (0,))) + _dg(lo, eye, ((0,), (0,)))


def _row_to_m(r, P, bmask8):
    return _xl(jnp.broadcast_to(r, (_B, _BC)) * bmask8, P)


def _x2(a, b):
    ah, al = _split(a)
    bh, bl = _split(b)
    return _dot(ah, bh) + _dot(ah, bl) + _dot(al, bh)


def _to_col(m, eexp, P):
    return jnp.sum(_xr(eexp, m) * P, axis=-1, keepdims=True)


def _summarize(c2d, sp, ma, bmask8, Q, eexp, eye):
    prec = 1.0 / (sp + _EPS)
    scores = ma + jnp.log(prec + _EPS)
    smax = jnp.max(scores, axis=-1, keepdims=True)
    e = jnp.exp(scores - smax)
    alpha = e / jnp.sum(e, axis=-1, keepdims=True)
    A = _xl(alpha, Q) * bmask8
    core = _dot(A, c2d)
    unc = jnp.sum(alpha * sp, axis=-1, keepdims=True)
    sqn = jnp.sum(c2d * c2d, axis=-1, keepdims=True)
    div = (_x2(A, sqn)
           - jnp.sum(core * core, axis=-1, keepdims=True)) * (1.0 / _H)
    mmax = jnp.max(ma, axis=-1, keepdims=True)
    en = jnp.log(jnp.sum(jnp.exp(ma - mmax), axis=-1, keepdims=True)) + mmax
    ent = -jnp.sum(alpha * jnp.log(jnp.maximum(alpha, 1e-8)),
                   axis=-1, keepdims=True)
    return core, unc, div, en, ent, alpha


def _scan_kernel(gi_ref, mask_ref, whhT_ref, bhh_ref,
                 w1lc_ref, w1s_ref, b1_ref, w2_ref, b2_ref,
                 hw_ref, hb_ref, ccb_ref, attb_ref,
                 o1m_ref, o1s_ref, ob1_ref,
                 h1_ref,
                 c2d_ref, sp_ref, ma_ref, h_ref, cnd_ref, core_ref, scal_ref,
                 alpha_ref, lp_ref):
    s = pl.program_id(0)
    bmask8, Q, P, eexp, eye, same = _consts()

    @pl.when(s == 0)
    def _init():
        c2d_ref[...] = jnp.zeros_like(c2d_ref)
        sp_ref[...] = jnp.ones_like(sp_ref)
        ma_ref[...] = jnp.zeros_like(ma_ref)
        core_ref[...] = jnp.zeros_like(core_ref)
        lc = math.log(_C)
        col8 = lax.broadcasted_iota(jnp.int32, (_B, 8), 1)
        scal_ref[...] = jnp.where(
            col8 == 0, 1.0, jnp.where((col8 == 2) | (col8 == 3), lc, 0.0))
        alpha_ref[...] = jnp.full((_B, _C), 1.0 / _C, jnp.float32)
        lp_ref[...] = jnp.zeros_like(lp_ref)
        gi0 = gi_ref[0:_B, :]
        gh0 = jnp.broadcast_to(bhh_ref[...], (_B, 3 * _H))
        r0 = jax.nn.sigmoid(gi0[:, :_H] + gh0[:, :_H])
        z0 = jax.nn.sigmoid(gi0[:, _H:2 * _H] + gh0[:, _H:2 * _H])
        n0 = jnp.tanh(gi0[:, 2 * _H:] + r0 * gh0[:, 2 * _H:])
        h_ref[...] = (1.0 - z0) * n0

    valid = mask_ref[0]
    local = h_ref[...]
    c2d = c2d_ref[...]
    sp = sp_ref[...]
    ma = ma_ref[...]
    core = core_ref[...]
    unc = scal_ref[:, 0:1]
    div = scal_ref[:, 1:2]
    en = scal_ref[:, 2:3]
    ent = scal_ref[:, 3:4]

    c2d0 = c2d

    pre = _dot(jnp.concatenate([local, core], axis=1).astype(jnp.bfloat16),
               w1lc_ref[...]) + b1_ref[...]
    pre = (pre + unc * w1s_ref[0:1, :] + div * w1s_ref[1:2, :]
           + en * w1s_ref[2:3, :] + ent * w1s_ref[3:4, :])
    ctrl = jnp.tanh(_dot(jnp.tanh(pre).astype(jnp.bfloat16), w2_ref[...])
                    + b2_ref[...])
    ctrl_bf = ctrl.astype(jnp.bfloat16)

    y1 = _dot(ctrl_bf, hw_ref[:, 0:2 * _H])
    hs = y1[:, 0:4 * _C + 2] + hb_ref[...]
    attr = y1[:, _H:2 * _H] + attb_ref[...]
    cand = _dot(ctrl_bf, hw_ref[:, 2 * _H:]) + ccb_ref[...]
    gate = jax.nn.sigmoid(hs[:, 0:_C]) * valid
    ae = hs[:, _C:2 * _C]
    ae = jnp.exp(ae - jnp.max(ae, axis=-1, keepdims=True))
    assign = ae / jnp.sum(ae, axis=-1, keepdims=True)
    cs_raw = hs[:, 2 * _C:3 * _C]
    cand_sp = (jnp.maximum(cs_raw, 0.0)
               + jnp.log(1.0 + jnp.exp(-jnp.abs(cs_raw))) + _EPS)
    mdel = jnp.tanh(hs[:, 3 * _C:4 * _C])
    nov = jax.nn.sigmoid(hs[:, 4 * _C:4 * _C + 1]) * valid
    relax = jax.nn.sigmoid(hs[:, 4 * _C + 1:4 * _C + 2]) * valid

    for c in range(_C):
        cnd_ref[c * _B:(c + 1) * _B, :] = cand[:, c * _H:(c + 1) * _H]
    cand2d = cnd_ref[...]

    strength = gate * assign
    st_col = _to_col(strength, eexp, P)
    c2d = c2d + st_col * (cand2d - c2d)
    sp = sp + strength * (cand_sp - sp)
    ma = ma + strength * mdel

    anr = _xr(eexp, jnp.concatenate([attr, nov, relax], axis=1))
    attr_exp = anr[:, :_H]
    nov_col = anr[:, _H:_H + 1]
    relax_col = anr[:, _H + 1:_H + 2]
    c2d = c2d + 0.1 * nov_col * (attr_exp - c2d)

    sp_row = _to_row(sp, Q, bmask8)
    ma_row = _to_row(ma, Q, bmask8)
    sp_col = _to_col(sp, eexp, P)
    sq_col = jnp.sum(c2d * c2d, axis=-1, keepdims=True)
    c2d_bf = c2d.astype(jnp.bfloat16)
    G = _dg(c2d_bf, c2d_bf, ((1,), (1,)))
    sq_row = jnp.sum(G * eye, axis=0, keepdims=True)
    d2 = jnp.maximum(sq_col + sq_row - 2.0 * G, 0.0)
    scale = sp_col + sp_row + _EPS
    compat = jnp.where(same, -d2 / scale + ma_row, _NEG)
    alpha2 = alpha_ref[...]
    cidx = lax.broadcasted_iota(jnp.int32, (_B, _C), 1)
    amax = jnp.max(alpha2, axis=-1, keepdims=True)
    idx = jnp.min(jnp.where(alpha2 == amax, cidx, _C), axis=-1, keepdims=True)
    jj = lax.broadcasted_iota(jnp.int32, (_B, _BC), 1)
    bb = lax.broadcasted_iota(jnp.int32, (_B, _BC), 0)
    oh = (((jj // _B) == idx) & ((jj % _B) == bb)).astype(jnp.float32)
    strongest = _dot(oh, c2d0)
    feat = jnp.concatenate([lp_ref[...], core, strongest], axis=1)
    h1 = _dot(feat.astype(jnp.bfloat16), o1m_ref[...]) + ob1_ref[...]
    h1 = (h1 + unc * o1s_ref[0:1, :] + div * o1s_ref[1:2, :]
          + en * o1s_ref[2:3, :] + ent * o1s_ref[3:4, :])
    h1 = 0.5 * h1 * (1.0 + lax.erf(h1 * _INVSQRT2))
    h1_ref[...] = h1

    cmax = jnp.max(compat, axis=-1, keepdims=True)
    cexp = jnp.exp(compat - cmax)
    mixing = cexp / jnp.sum(cexp, axis=-1, keepdims=True)
    mc = _dot(mixing.astype(jnp.bfloat16), c2d_bf)
    msp = jnp.sum(mixing * sp_row, axis=-1, keepdims=True)
    mma = jnp.sum(mixing * ma_row, axis=-1, keepdims=True)

    c2d = (1.0 - relax_col) * c2d + relax_col * mc
    mrows = _col_to_row(jnp.concatenate([msp, mma], axis=1), eye)
    sp = (1.0 - relax) * sp + relax * _row_to_m(mrows[0:1], P, bmask8)
    ma = (1.0 - relax) * ma + relax * _row_to_m(mrows[1:2], P, bmask8)

    c2d_ref[...] = c2d
    sp_ref[...] = sp
    ma_ref[...] = ma
    lp_ref[...] = local

    sn = jnp.minimum(s + 1, _S - 1)
    gi = gi_ref[pl.ds(pl.multiple_of(sn * _B, _B), _B), :]
    gh = _dg(local.astype(jnp.bfloat16), whhT_ref[...], ((1,), (1,))) + bhh_ref[...]
    r = jax.nn.sigmoid(gi[:, :_H] + gh[:, :_H])
    z = jax.nn.sigmoid(gi[:, _H:2 * _H] + gh[:, _H:2 * _H])
    n = jnp.tanh(gi[:, 2 * _H:] + r * gh[:, 2 * _H:])
    h_ref[...] = (1.0 - z) * n + z * local


    core2, unc2, div2, en2, ent2, alpha2 = _summarize(
        c2d, sp, ma, bmask8, Q, eexp, eye)
    core_ref[...] = core2
    scal_ref[...] = jnp.concatenate(
        [unc2, div2, en2, ent2, jnp.zeros((_B, 4), jnp.float32)], axis=1)
    alpha_ref[...] = alpha2


def _logits_kernel(a_ref, w_ref, b_ref, o_ref):
    o_ref[...] = _dot(a_ref[...], w_ref[...]) + b_ref[...]


def _gi_kernel(x_ref, wih_ref, b_ref, hw_ref, att_ref, cc_ref, whh_ref,
               w1_ref, w2_ref, o1_ref,
               gi_ref, wide_ref, whh_o, w1_o, w2_o, o1_o):
    gi_ref[...] = _dg(x_ref[...], wih_ref[...], ((1,), (1,))) + b_ref[...]
    bf = jnp.bfloat16
    wide_ref[:, 0:4 * _C + 2] = hw_ref[...].astype(bf)
    wide_ref[:, 4 * _C + 2:_H] = jnp.zeros((_H, _H - (4 * _C + 2)), bf)
    wide_ref[:, _H:2 * _H] = att_ref[...].astype(bf)
    wide_ref[:, 2 * _H:] = cc_ref[...].astype(bf)
    whh_o[...] = whh_ref[...].astype(bf)
    w1_o[...] = w1_ref[0:2 * _H, :].astype(bf)
    w2_o[...] = w2_ref[...].astype(bf)
    o1_o[...] = o1_ref[0:3 * _H, :].astype(bf)


_NT = 3200


@functools.partial(jax.jit, static_argnames=("interpret",))
def _run(x_flat, maskS, wihT, bih2, whhT, bhh2, w1lc, w1s, b1_2, ctrl_w2,
         b2_2, hw, hb, cc_w, cc_b2, att_w, attb2, o1m, o1s, ob1, out_w2,
         ob2, interpret=False):
    const = lambda s: (0, 0)
    bf = jnp.bfloat16
    gi_all, hw_wide, whh_bf, w1lc_bf, w2_bf, o1m_bf = pl.pallas_call(
        _gi_kernel,
        out_shape=[jax.ShapeDtypeStruct((_S * _B, 3 * _H), jnp.float32),
                   jax.ShapeDtypeStruct((_H, 2 * _H + _C * _H), jnp.bfloat16),
                   jax.ShapeDtypeStruct((3 * _H, _H), jnp.bfloat16),
                   jax.ShapeDtypeStruct((2 * _H, _H), jnp.bfloat16),
                   jax.ShapeDtypeStruct((_H, _H), jnp.bfloat16),
                   jax.ShapeDtypeStruct((3 * _H, _H), jnp.bfloat16)],
        name="cfrm_gi",
        interpret=interpret,
    )(x_flat, wihT, bih2, hw, att_w, cc_w, whhT, w1lc, ctrl_w2, o1m)

    h1_sb = pl.pallas_call(
        _scan_kernel,
        out_shape=jax.ShapeDtypeStruct((_B, _S * _H), jnp.float32),
        grid=(_S + 1,),
        in_specs=[
            pl.BlockSpec((_S * _B, 3 * _H), const),
            pl.BlockSpec((1, _B, 1),
                         lambda s: (jnp.minimum(s, _S - 1), 0, 0)),
            pl.BlockSpec((3 * _H, _H), const),
            pl.BlockSpec((1, 3 * _H), const),
            pl.BlockSpec((2 * _H, _H), const),
            pl.BlockSpec((4, _H), const),
            pl.BlockSpec((1, _H), const),
            pl.BlockSpec((_H, _H), const),
            pl.BlockSpec((1, _H), const),
            pl.BlockSpec((_H, 2 * _H + _C * _H), const),
            pl.BlockSpec((1, 4 * _C + 2), const),
            pl.BlockSpec((1, _C * _H), const),
            pl.BlockSpec((1, _H), const),
            pl.BlockSpec((3 * _H, _H), const),
            pl.BlockSpec((4, _H), const),
            pl.BlockSpec((1, _H), const),
        ],
        out_specs=pl.BlockSpec((_B, _H),
                               lambda s: (0, jnp.maximum(s - 1, 0))),
        scratch_shapes=[
            pltpu.VMEM((_BC, _H), jnp.float32),
            pltpu.VMEM((_B, _C), jnp.float32),
            pltpu.VMEM((_B, _C), jnp.float32),
            pltpu.VMEM((_B, _H), jnp.float32),
            pltpu.VMEM((_BC, _H), jnp.float32),
            pltpu.VMEM((_B, _H), jnp.float32),
            pltpu.VMEM((_B, 8), jnp.float32),
            pltpu.VMEM((_B, _C), jnp.float32),
            pltpu.VMEM((_B, _H), jnp.float32),
        ],
        compiler_params=pltpu.CompilerParams(
            dimension_semantics=("arbitrary",),
            vmem_limit_bytes=50 * 1024 * 1024,
        ),
        name="cfrm_scan",
        interpret=interpret,
    )(gi_all, maskS, whh_bf, bhh2, w1lc_bf, w1s, b1_2,
      w2_bf, b2_2, hw_wide, hb, cc_b2,
      attb2, o1m_bf, o1s, ob1)

    h1_bs = h1_sb.reshape(_B * _S, _H)
    logits = pl.pallas_call(
        _logits_kernel,
        out_shape=jax.ShapeDtypeStruct((_B * _S, _V), jnp.float32),
        grid=(_V // _NT,),
        in_specs=[
            pl.BlockSpec((_B * _S, _H), lambda j: (0, 0)),
            pl.BlockSpec((_H, _NT), lambda j: (0, j)),
            pl.BlockSpec((1, _NT), lambda j: (0, j)),
        ],
        out_specs=pl.BlockSpec((_B * _S, _NT), lambda j: (0, j)),
        compiler_params=pltpu.CompilerParams(
            dimension_semantics=("parallel",),
            vmem_limit_bytes=48 * 1024 * 1024,
        ),
        name="cfrm_logits",
        interpret=interpret,
    )(h1_bs.astype(bf), out_w2.astype(bf), ob2)
    return logits.reshape(_B, _S, _V)


def kernel(tokens, emb, gru_wih, gru_whh, gru_bih, gru_bhh, ctrl_w1, ctrl_b1,
           ctrl_w2, ctrl_b2, gate_w, gate_b, assign_w, assign_b, nov_w, nov_b,
           relax_w, relax_b, cc_w, cc_b, cs_w, cs_b, md_w, md_b, att_w, att_b,
           out_w1, out_b1, out_w2, out_b2, *, interpret=False):
    x = emb[tokens]
    x_flat = jnp.swapaxes(x, 0, 1).reshape(_S * _B, _E)
    maskS = (tokens != 0).astype(jnp.float32).T[:, :, None]
    hw = jnp.concatenate([gate_w, assign_w, cs_w, md_w, nov_w, relax_w], axis=1)
    hb = jnp.concatenate([gate_b, assign_b, cs_b, md_b, nov_b, relax_b])[None, :]
    return _run(x_flat, maskS, gru_wih, gru_bih[None, :], gru_whh,
                gru_bhh[None, :], ctrl_w1, ctrl_w1[2 * _H:],
                ctrl_b1[None, :], ctrl_w2, ctrl_b2[None, :], hw, hb,
                cc_w, cc_b[None, :], att_w, att_b[None, :],
                out_w1, out_w1[3 * _H:], out_b1[None, :],
                out_w2, out_b2[None, :], interpret=interpret)
